# baseline (speedup 1.0000x reference)
import functools

import jax
import jax.numpy as jnp
from jax import lax
from jax.experimental import pallas as pl
from jax.experimental.pallas import tpu as pltpu
from jax.experimental.pallas import tpu_sc as plsc

N_TEAMS = 100000
D = 32
H = 64
E = 20000
B = 16384
PASSES = 3

EP = 20480
TOT = 2 * EP
NW = 32
NSUB = 16
HALF = 50000
DUMP = HALF
SUMROWS = HALF + 8
OUT_DUMP = TOT
XDUMP = N_TEAMS
XROWS = N_TEAMS + 8
CW = 128
NCH = TOT // NW // CW
CHUNK = NCH * CW
SEG_CH = TOT // NSUB // CW
SEG_ROWS = SEG_CH * CW

_MESH = plsc.VectorSubcoreMesh(core_axis_name="c", subcore_axis_name="s")

BLK = 2048
NB = TOT // BLK
HB = EP // BLK


def _make_sc_gather(nch):
    rows_per_tile = nch * CW

    @functools.partial(
        pl.kernel,
        out_type=jax.ShapeDtypeStruct((NW * rows_per_tile, D), jnp.float32),
        mesh=_MESH,
        compiler_params=pltpu.CompilerParams(use_tc_tiling_on_sc=False),
        scratch_types=[
            pltpu.VMEM((nch, CW), jnp.int32),
            pltpu.VMEM((rows_per_tile, D), jnp.float32),
            pltpu.SemaphoreType.DMA,
        ],
    )
    def k(table, idx3, out, idx_v, buf, sem):
        c = lax.axis_index("c")
        s = lax.axis_index("s")
        w = c * NSUB + s
        pltpu.sync_copy(idx3.at[w], idx_v)

        def body(kk, _):
            pltpu.async_copy(table.at[idx_v.at[kk]],
                             buf.at[pl.ds(kk * CW, CW)], sem).wait()
            return 0

        lax.fori_loop(0, nch, body, 0)
        pltpu.sync_copy(buf, out.at[pl.ds(w * rows_per_tile, rows_per_tile)])

    return k


_sc_gather_edges = _make_sc_gather(NCH)
_sc_gather_pairs = _make_sc_gather(2 * B // NW // CW)


@functools.partial(
    pl.kernel,
    out_type=jax.ShapeDtypeStruct((TOT + 8, D), jnp.float32),
    mesh=_MESH,
    compiler_params=pltpu.CompilerParams(use_tc_tiling_on_sc=False),
    scratch_types=[
        pltpu.VMEM_SHARED((SUMROWS, D), jnp.float32),
        pltpu.VMEM((SEG_CH, CW), jnp.int32),
        pltpu.VMEM((SEG_CH, CW), jnp.int32),
        pltpu.VMEM((CW, D), jnp.float32),
        pltpu.SemaphoreType.DMA,
    ],
)
def _sc_segsum(msgs, zrows, tgt_all, pos_all, out, sums, idx_v, pos_v, mbuf, sem):
    c = lax.axis_index("c")
    s = lax.axis_index("s")
    pltpu.sync_copy(tgt_all.at[c, s], idx_v)
    pltpu.sync_copy(pos_all.at[c, s], pos_v)
    pltpu.sync_copy(zrows, mbuf)

    def zero_body(kk, _):
        pltpu.sync_copy(mbuf, sums.at[idx_v.at[kk]])
        return 0

    lax.fori_loop(0, SEG_CH, zero_body, 0)
    plsc.subcore_barrier()

    def add_body(kk, _):
        pltpu.sync_copy(msgs.at[pl.ds(s * SEG_ROWS + kk * CW, CW)], mbuf)
        pltpu.sync_copy(mbuf, sums.at[idx_v.at[kk]], add=True)
        return 0

    lax.fori_loop(0, SEG_CH, add_body, 0)
    plsc.subcore_barrier()

    def back_body(kk, _):
        pltpu.async_copy(sums.at[idx_v.at[kk]], mbuf, sem).wait()
        pltpu.async_copy(mbuf, out.at[pos_v.at[kk]], sem).wait()
        return 0

    lax.fori_loop(0, SEG_CH, back_body, 0)


@functools.partial(
    pl.kernel,
    out_type=jax.ShapeDtypeStruct((XROWS, D), jnp.float32),
    mesh=_MESH,
    compiler_params=pltpu.CompilerParams(use_tc_tiling_on_sc=False),
    scratch_types=[
        pltpu.VMEM((625, D), jnp.float32),
        pltpu.VMEM((SEG_CH, CW), jnp.int32),
        pltpu.VMEM((CW, D), jnp.float32),
        pltpu.SemaphoreType.DMA,
    ],
)
def _sc_buildx(emb, upd, xtgt_all, x, cbuf, idx_v, mbuf, sem):
    c = lax.axis_index("c")
    s = lax.axis_index("s")
    base = c * HALF + s * (HALF // NSUB)

    def copy_body(kk, _):
        off = base + kk * 625
        pltpu.sync_copy(emb.at[pl.ds(off, 625)], cbuf)
        pltpu.sync_copy(cbuf, x.at[pl.ds(off, 625)])
        return 0

    lax.fori_loop(0, (HALF // NSUB) // 625, copy_body, 0)
    plsc.subcore_barrier()

    pltpu.sync_copy(xtgt_all.at[c, s], idx_v)

    def scat_body(kk, _):
        pltpu.sync_copy(upd.at[pl.ds(s * SEG_ROWS + kk * CW, CW)], mbuf)
        pltpu.async_copy(mbuf, x.at[idx_v.at[kk]], sem).wait()
        return 0

    lax.fori_loop(0, SEG_CH, scat_body, 0)


def _spec_a():
    return pl.BlockSpec((BLK, D), lambda i: (i, 0))


def _spec_b():
    return pl.BlockSpec((BLK, D), lambda i: ((i + HB) % NB, 0))


def _wspec(shape):
    nd = len(shape)
    return pl.BlockSpec(shape, lambda i: (0,) * nd)


def _tc_msgs_body(xa, xb, w1, b1, w2, b2, o):
    a = jnp.concatenate([xa[...], xb[...]], axis=1)
    h = jnp.maximum(jnp.dot(a, w1[...], preferred_element_type=jnp.float32)
                    + b1[...], 0.0)
    o[...] = jnp.dot(h, w2[...], preferred_element_type=jnp.float32) + b2[...]


def _tc_msgs(rows, W1, b1, W2, b2, swapped):
    first, second = (_spec_b(), _spec_a()) if swapped else (_spec_a(), _spec_b())
    return pl.pallas_call(
        _tc_msgs_body,
        grid=(NB,),
        in_specs=[first, second, _wspec((2 * D, H)), _wspec((1, H)),
                  _wspec((H, D)), _wspec((1, D))],
        out_specs=_spec_a(),
        out_shape=jax.ShapeDtypeStruct((TOT, D), jnp.float32),
    )(rows, rows, W1, b1, W2, b2)


def _tc_upd_body(xt, sv, cv, wu, bu, o):
    mean = sv[...] / jnp.maximum(cv[...], 1.0)
    a = jnp.concatenate([xt[...], mean], axis=1)
    o[...] = jnp.maximum(
        jnp.dot(a, wu[...], preferred_element_type=jnp.float32) + bu[...], 0.0)


def _tc_upd(rows, sv, cv, Wu, bu, swapped):
    xt_spec = _spec_a() if swapped else _spec_b()
    return pl.pallas_call(
        _tc_upd_body,
        grid=(NB,),
        in_specs=[xt_spec, _spec_a(), _spec_a(), _wspec((2 * D, D)),
                  _wspec((1, D))],
        out_specs=_spec_a(),
        out_shape=jax.ShapeDtypeStruct((TOT, D), jnp.float32),
    )(rows, sv, cv, Wu, bu)


def _tc_pred_body(he, ae, w1, b1, w2, b2, w3, b3, o):
    a = jnp.concatenate([he[...], ae[...]], axis=1)
    p = jnp.maximum(jnp.dot(a, w1[...], preferred_element_type=jnp.float32)
                    + b1[...], 0.0)
    p = jnp.maximum(jnp.dot(p, w2[...], preferred_element_type=jnp.float32)
                    + b2[...], 0.0)
    z = jnp.dot(p, w3[...], preferred_element_type=jnp.float32) + b3[...]
    o[...] = 1.0 / (1.0 + jnp.exp(-z))


def _tc_pred(rows_ha, Wp1, bp1, Wp2, bp2, Wp3, bp3):
    nb = B // BLK
    he_spec = pl.BlockSpec((BLK, D), lambda i: (i, 0))
    ae_spec = pl.BlockSpec((BLK, D), lambda i: (i + nb, 0))
    return pl.pallas_call(
        _tc_pred_body,
        grid=(nb,),
        in_specs=[he_spec, ae_spec, _wspec((2 * D, H)), _wspec((1, H)),
                  _wspec((H, D)), _wspec((1, D)), _wspec((D, 1)),
                  _wspec((1, 1))],
        out_specs=pl.BlockSpec((BLK, 1), lambda i: (i, 0)),
        out_shape=jax.ShapeDtypeStruct((B, 1), jnp.float32),
    )(rows_ha, rows_ha, Wp1, bp1, Wp2, bp2, Wp3, bp3)


def kernel(emb, W1, b1, W2, b2, Wu, bu, Wp1, bp1, Wp2, bp2, Wp3, bp3,
           home_ids, away_ids, adjacency):
    adjacency = adjacency.astype(jnp.int32)
    home_ids = home_ids.astype(jnp.int32)
    away_ids = away_ids.astype(jnp.int32)
    src = adjacency[:, 0]
    dst = adjacency[:, 1]

    pad0 = jnp.zeros((EP - E,), jnp.int32)
    padm = jnp.full((EP - E,), -1, jnp.int32)
    idxcat = jnp.concatenate([src, pad0, dst, pad0])
    tgt = jnp.concatenate([dst, padm, src, padm])

    in0 = (tgt >= 0) & (tgt < HALF)
    in1 = tgt >= HALF
    tgt_all = jnp.stack([jnp.where(in0, tgt, DUMP),
                         jnp.where(in1, tgt - HALF, DUMP)])
    tgt_all = tgt_all.reshape(2, NSUB, SEG_CH, CW)
    pos = jnp.arange(TOT, dtype=jnp.int32)
    pos_all = jnp.stack([jnp.where(in0, pos, OUT_DUMP),
                         jnp.where(in1, pos, OUT_DUMP)])
    pos_all = pos_all.reshape(2, NSUB, SEG_CH, CW)
    xtgt_all = jnp.stack([jnp.where(in0, tgt, XDUMP),
                          jnp.where(in1, tgt, XDUMP)])
    xtgt_all = xtgt_all.reshape(2, NSUB, SEG_CH, CW)
    idx3 = idxcat.reshape(NW, NCH, CW)
    ha3 = jnp.concatenate([home_ids, away_ids]).reshape(NW, 2 * B // NW // CW, CW)

    zrows = jnp.zeros((CW, D), jnp.float32)
    ones = jnp.ones((TOT, D), jnp.float32)
    b1r = b1.reshape(1, H)
    b2r = b2.reshape(1, D)
    bur = bu.reshape(1, D)
    bp1r = bp1.reshape(1, H)
    bp2r = bp2.reshape(1, D)
    bp3r = bp3.reshape(1, 1)

    rows = _sc_gather_edges(emb, idx3)
    cv = _sc_segsum(ones, zrows, tgt_all, pos_all)[:TOT]

    swapped = False
    for _ in range(PASSES):
        msgs = _tc_msgs(rows, W1, b1r, W2, b2r, swapped)
        sv = _sc_segsum(msgs, zrows, tgt_all, pos_all)[:TOT]
        rows = _tc_upd(rows, sv, cv, Wu, bur, swapped)
        swapped = True

    x = _sc_buildx(emb, rows, xtgt_all)
    rows_ha = _sc_gather_pairs(x, ha3)
    return _tc_pred(rows_ha, Wp1, bp1r, Wp2, bp2r, Wp3, bp3r)


# speedup vs baseline: 1.0050x; 1.0050x over previous
import functools

import jax
import jax.numpy as jnp
from jax import lax
from jax.experimental import pallas as pl
from jax.experimental.pallas import tpu as pltpu
from jax.experimental.pallas import tpu_sc as plsc

N_TEAMS = 100000
D = 32
H = 64
E = 20000
B = 16384
PASSES = 3

EP = 20480
TOT = 2 * EP
NW = 32
NSUB = 16
HALF = 50000
DUMP = HALF
SUMROWS = HALF + 8
OUT_DUMP = TOT
XDUMP = N_TEAMS
XROWS = N_TEAMS + 8
CW = 128
NCH = TOT // NW // CW
CHUNK = NCH * CW
SEG_CH = TOT // NSUB // CW
SEG_ROWS = SEG_CH * CW
RB = 5
NG = SEG_CH // RB

_MESH = plsc.VectorSubcoreMesh(core_axis_name="c", subcore_axis_name="s")

BLK = 2048
NB = TOT // BLK
HB = EP // BLK


def _make_sc_gather(nch):
    rows_per_tile = nch * CW

    @functools.partial(
        pl.kernel,
        out_type=jax.ShapeDtypeStruct((NW * rows_per_tile, D), jnp.float32),
        mesh=_MESH,
        compiler_params=pltpu.CompilerParams(use_tc_tiling_on_sc=False),
        scratch_types=[
            pltpu.VMEM((nch, CW), jnp.int32),
            pltpu.VMEM((rows_per_tile, D), jnp.float32),
            pltpu.SemaphoreType.DMA,
        ],
    )
    def k(table, idx3, out, idx_v, buf, sem):
        c = lax.axis_index("c")
        s = lax.axis_index("s")
        w = c * NSUB + s
        pltpu.sync_copy(idx3.at[w], idx_v)

        def fire(kk, _):
            pltpu.async_copy(table.at[idx_v.at[kk]],
                             buf.at[pl.ds(kk * CW, CW)], sem)
            return 0

        lax.fori_loop(0, nch, fire, 0)

        def drain(kk, _):
            pltpu.make_async_copy(table.at[idx_v.at[0]],
                                  buf.at[pl.ds(0, CW)], sem).wait()
            return 0

        lax.fori_loop(0, nch, drain, 0)
        pltpu.sync_copy(buf, out.at[pl.ds(w * rows_per_tile, rows_per_tile)])

    return k


_sc_gather_edges = _make_sc_gather(NCH)
_sc_gather_pairs = _make_sc_gather(2 * B // NW // CW)


@functools.partial(
    pl.kernel,
    out_type=jax.ShapeDtypeStruct((TOT + 8, D), jnp.float32),
    mesh=_MESH,
    compiler_params=pltpu.CompilerParams(use_tc_tiling_on_sc=False),
    scratch_types=[
        pltpu.VMEM_SHARED((SUMROWS, D), jnp.float32),
        pltpu.VMEM((SEG_CH, CW), jnp.int32),
        pltpu.VMEM((SEG_CH, CW), jnp.int32),
        pltpu.VMEM((RB, CW, D), jnp.float32),
        pltpu.SemaphoreType.DMA((RB,)),
        pltpu.SemaphoreType.DMA((RB,)),
    ],
)
def _sc_segsum(msgs, zrows, tgt_all, pos_all, out, sums, idx_v, pos_v, ring,
               semL, semA):
    c = lax.axis_index("c")
    s = lax.axis_index("s")
    pltpu.sync_copy(tgt_all.at[c, s], idx_v)
    pltpu.sync_copy(pos_all.at[c, s], pos_v)
    base = s * SEG_ROWS

    pltpu.sync_copy(zrows, ring.at[0])

    def zfire(kk, _):
        pltpu.async_copy(ring.at[0], sums.at[idx_v.at[kk]], semA.at[0])
        return 0

    lax.fori_loop(0, SEG_CH, zfire, 0)

    def zdrain(kk, _):
        pltpu.make_async_copy(ring.at[0], sums.at[idx_v.at[0]],
                              semA.at[0]).wait()
        return 0

    lax.fori_loop(0, SEG_CH, zdrain, 0)
    plsc.subcore_barrier()

    def agroup(g, _):
        for b in range(RB):
            k = g * RB + b

            @pl.when(g > 0)
            def _wait_add():
                pltpu.make_async_copy(ring.at[b], sums.at[idx_v.at[0]],
                                      semA.at[b]).wait()

            pltpu.async_copy(msgs.at[pl.ds(base + k * CW, CW)], ring.at[b],
                             semL.at[b])
        for b in range(RB):
            k = g * RB + b
            pltpu.make_async_copy(msgs.at[pl.ds(base, CW)], ring.at[b],
                                  semL.at[b]).wait()
            pltpu.async_copy(ring.at[b], sums.at[idx_v.at[k]], semA.at[b],
                             add=True)
        return 0

    lax.fori_loop(0, NG, agroup, 0)
    for b in range(RB):
        pltpu.make_async_copy(ring.at[b], sums.at[idx_v.at[0]],
                              semA.at[b]).wait()
    plsc.subcore_barrier()

    def bgroup(g, _):
        for b in range(RB):
            k = g * RB + b

            @pl.when(g > 0)
            def _wait_out():
                pltpu.make_async_copy(ring.at[b], out.at[pos_v.at[0]],
                                      semA.at[b]).wait()

            pltpu.async_copy(sums.at[idx_v.at[k]], ring.at[b], semL.at[b])
        for b in range(RB):
            k = g * RB + b
            pltpu.make_async_copy(sums.at[idx_v.at[0]], ring.at[b],
                                  semL.at[b]).wait()
            pltpu.async_copy(ring.at[b], out.at[pos_v.at[k]], semA.at[b])
        return 0

    lax.fori_loop(0, NG, bgroup, 0)
    for b in range(RB):
        pltpu.make_async_copy(ring.at[b], out.at[pos_v.at[0]],
                              semA.at[b]).wait()


@functools.partial(
    pl.kernel,
    out_type=jax.ShapeDtypeStruct((XROWS, D), jnp.float32),
    mesh=_MESH,
    compiler_params=pltpu.CompilerParams(use_tc_tiling_on_sc=False),
    scratch_types=[
        pltpu.VMEM((HALF // NSUB, D), jnp.float32),
        pltpu.VMEM((SEG_CH, CW), jnp.int32),
        pltpu.VMEM((RB, CW, D), jnp.float32),
        pltpu.SemaphoreType.DMA((RB,)),
        pltpu.SemaphoreType.DMA((RB,)),
    ],
)
def _sc_buildx(emb, upd, xtgt_all, x, cbuf, idx_v, ring, semL, semA):
    c = lax.axis_index("c")
    s = lax.axis_index("s")
    tbase = c * HALF + s * (HALF // NSUB)
    pltpu.sync_copy(emb.at[pl.ds(tbase, HALF // NSUB)], cbuf)
    pltpu.sync_copy(cbuf, x.at[pl.ds(tbase, HALF // NSUB)])
    plsc.subcore_barrier()

    pltpu.sync_copy(xtgt_all.at[c, s], idx_v)
    base = s * SEG_ROWS

    def group(g, _):
        for b in range(RB):
            k = g * RB + b

            @pl.when(g > 0)
            def _wait_scat():
                pltpu.make_async_copy(ring.at[b], x.at[idx_v.at[0]],
                                      semA.at[b]).wait()

            pltpu.async_copy(upd.at[pl.ds(base + k * CW, CW)], ring.at[b],
                             semL.at[b])
        for b in range(RB):
            k = g * RB + b
            pltpu.make_async_copy(upd.at[pl.ds(base, CW)], ring.at[b],
                                  semL.at[b]).wait()
            pltpu.async_copy(ring.at[b], x.at[idx_v.at[k]], semA.at[b])
        return 0

    lax.fori_loop(0, NG, group, 0)
    for b in range(RB):
        pltpu.make_async_copy(ring.at[b], x.at[idx_v.at[0]], semA.at[b]).wait()


def _spec_a():
    return pl.BlockSpec((BLK, D), lambda i: (i, 0))


def _spec_b():
    return pl.BlockSpec((BLK, D), lambda i: ((i + HB) % NB, 0))


def _wspec(shape):
    nd = len(shape)
    return pl.BlockSpec(shape, lambda i: (0,) * nd)


def _tc_msgs_body(xa, xb, w1, b1, w2, b2, o):
    a = jnp.concatenate([xa[...], xb[...]], axis=1)
    h = jnp.maximum(jnp.dot(a, w1[...], preferred_element_type=jnp.float32)
                    + b1[...], 0.0)
    o[...] = jnp.dot(h, w2[...], preferred_element_type=jnp.float32) + b2[...]


def _tc_msgs(rows, W1, b1, W2, b2, swapped):
    first, second = (_spec_b(), _spec_a()) if swapped else (_spec_a(), _spec_b())
    return pl.pallas_call(
        _tc_msgs_body,
        grid=(NB,),
        in_specs=[first, second, _wspec((2 * D, H)), _wspec((1, H)),
                  _wspec((H, D)), _wspec((1, D))],
        out_specs=_spec_a(),
        out_shape=jax.ShapeDtypeStruct((TOT, D), jnp.float32),
    )(rows, rows, W1, b1, W2, b2)


def _tc_upd_body(xt, sv, cv, wu, bu, o):
    mean = sv[...] / jnp.maximum(cv[...], 1.0)
    a = jnp.concatenate([xt[...], mean], axis=1)
    o[...] = jnp.maximum(
        jnp.dot(a, wu[...], preferred_element_type=jnp.float32) + bu[...], 0.0)


def _tc_upd(rows, sv, cv, Wu, bu, swapped):
    xt_spec = _spec_a() if swapped else _spec_b()
    return pl.pallas_call(
        _tc_upd_body,
        grid=(NB,),
        in_specs=[xt_spec, _spec_a(), _spec_a(), _wspec((2 * D, D)),
                  _wspec((1, D))],
        out_specs=_spec_a(),
        out_shape=jax.ShapeDtypeStruct((TOT, D), jnp.float32),
    )(rows, sv, cv, Wu, bu)


def _tc_pred_body(he, ae, w1, b1, w2, b2, w3, b3, o):
    a = jnp.concatenate([he[...], ae[...]], axis=1)
    p = jnp.maximum(jnp.dot(a, w1[...], preferred_element_type=jnp.float32)
                    + b1[...], 0.0)
    p = jnp.maximum(jnp.dot(p, w2[...], preferred_element_type=jnp.float32)
                    + b2[...], 0.0)
    z = jnp.dot(p, w3[...], preferred_element_type=jnp.float32) + b3[...]
    o[...] = 1.0 / (1.0 + jnp.exp(-z))


def _tc_pred(rows_ha, Wp1, bp1, Wp2, bp2, Wp3, bp3):
    nb = B // BLK
    he_spec = pl.BlockSpec((BLK, D), lambda i: (i, 0))
    ae_spec = pl.BlockSpec((BLK, D), lambda i: (i + nb, 0))
    return pl.pallas_call(
        _tc_pred_body,
        grid=(nb,),
        in_specs=[he_spec, ae_spec, _wspec((2 * D, H)), _wspec((1, H)),
                  _wspec((H, D)), _wspec((1, D)), _wspec((D, 1)),
                  _wspec((1, 1))],
        out_specs=pl.BlockSpec((BLK, 1), lambda i: (i, 0)),
        out_shape=jax.ShapeDtypeStruct((B, 1), jnp.float32),
    )(rows_ha, rows_ha, Wp1, bp1, Wp2, bp2, Wp3, bp3)


def kernel(emb, W1, b1, W2, b2, Wu, bu, Wp1, bp1, Wp2, bp2, Wp3, bp3,
           home_ids, away_ids, adjacency):
    adjacency = adjacency.astype(jnp.int32)
    home_ids = home_ids.astype(jnp.int32)
    away_ids = away_ids.astype(jnp.int32)
    src = adjacency[:, 0]
    dst = adjacency[:, 1]

    pad0 = jnp.zeros((EP - E,), jnp.int32)
    padm = jnp.full((EP - E,), -1, jnp.int32)
    idxcat = jnp.concatenate([src, pad0, dst, pad0])
    tgt = jnp.concatenate([dst, padm, src, padm])

    in0 = (tgt >= 0) & (tgt < HALF)
    in1 = tgt >= HALF
    tgt_all = jnp.stack([jnp.where(in0, tgt, DUMP),
                         jnp.where(in1, tgt - HALF, DUMP)])
    tgt_all = tgt_all.reshape(2, NSUB, SEG_CH, CW)
    pos = jnp.arange(TOT, dtype=jnp.int32)
    pos_all = jnp.stack([jnp.where(in0, pos, OUT_DUMP),
                         jnp.where(in1, pos, OUT_DUMP)])
    pos_all = pos_all.reshape(2, NSUB, SEG_CH, CW)
    xtgt_all = jnp.stack([jnp.where(in0, tgt, XDUMP),
                          jnp.where(in1, tgt, XDUMP)])
    xtgt_all = xtgt_all.reshape(2, NSUB, SEG_CH, CW)
    idx3 = idxcat.reshape(NW, NCH, CW)
    ha3 = jnp.concatenate([home_ids, away_ids]).reshape(NW, 2 * B // NW // CW, CW)

    zrows = jnp.zeros((CW, D), jnp.float32)
    ones = jnp.ones((TOT, D), jnp.float32)
    b1r = b1.reshape(1, H)
    b2r = b2.reshape(1, D)
    bur = bu.reshape(1, D)
    bp1r = bp1.reshape(1, H)
    bp2r = bp2.reshape(1, D)
    bp3r = bp3.reshape(1, 1)

    rows = _sc_gather_edges(emb, idx3)
    cv = _sc_segsum(ones, zrows, tgt_all, pos_all)[:TOT]

    swapped = False
    for _ in range(PASSES):
        msgs = _tc_msgs(rows, W1, b1r, W2, b2r, swapped)
        sv = _sc_segsum(msgs, zrows, tgt_all, pos_all)[:TOT]
        rows = _tc_upd(rows, sv, cv, Wu, bur, swapped)
        swapped = True

    x = _sc_buildx(emb, rows, xtgt_all)
    rows_ha = _sc_gather_pairs(x, ha3)
    return _tc_pred(rows_ha, Wp1, bp1r, Wp2, bp2r, Wp3, bp3r)


# speedup vs baseline: 3.9378x; 3.9181x over previous
import functools

import jax
import jax.numpy as jnp
from jax import lax
from jax.experimental import pallas as pl
from jax.experimental.pallas import tpu as pltpu
from jax.experimental.pallas import tpu_sc as plsc

N_TEAMS = 100000
D = 32
H = 64
E = 20000
B = 16384
PASSES = 3

EP = 20480
TOT = 2 * EP
NW = 32
NSUB = 16
HALF = 50000
NSLOP = 64
DUMP = HALF
SUMROWS = HALF + NSLOP
OUT_DUMP = TOT
XDUMP = N_TEAMS
XROWS = N_TEAMS + NSLOP
CW = 128
NCH = TOT // NW // CW
CHUNK = NCH * CW
SEG_CH = TOT // NSUB // CW
SEG_ROWS = SEG_CH * CW
RB = 5
NG = SEG_CH // RB

_MESH = plsc.VectorSubcoreMesh(core_axis_name="c", subcore_axis_name="s")

BLK = 2048
NB = TOT // BLK
HB = EP // BLK


def _make_sc_gather(nch):
    rows_per_tile = nch * CW

    @functools.partial(
        pl.kernel,
        out_type=jax.ShapeDtypeStruct((NW * rows_per_tile, D), jnp.float32),
        mesh=_MESH,
        compiler_params=pltpu.CompilerParams(use_tc_tiling_on_sc=False),
        scratch_types=[
            pltpu.VMEM((nch, CW), jnp.int32),
            pltpu.VMEM((rows_per_tile, D), jnp.float32),
            pltpu.SemaphoreType.DMA,
        ],
    )
    def k(table, idx3, out, idx_v, buf, sem):
        c = lax.axis_index("c")
        s = lax.axis_index("s")
        w = c * NSUB + s
        pltpu.sync_copy(idx3.at[w], idx_v)

        def fire(kk, _):
            pltpu.async_copy(table.at[idx_v.at[kk]],
                             buf.at[pl.ds(kk * CW, CW)], sem)
            return 0

        lax.fori_loop(0, nch, fire, 0)

        def drain(kk, _):
            pltpu.make_async_copy(table.at[idx_v.at[0]],
                                  buf.at[pl.ds(0, CW)], sem).wait()
            return 0

        lax.fori_loop(0, nch, drain, 0)
        pltpu.sync_copy(buf, out.at[pl.ds(w * rows_per_tile, rows_per_tile)])

    return k


_sc_gather_edges = _make_sc_gather(NCH)
_sc_gather_pairs = _make_sc_gather(2 * B // NW // CW)


@functools.partial(
    pl.kernel,
    out_type=jax.ShapeDtypeStruct((TOT + NSLOP, D), jnp.float32),
    mesh=_MESH,
    compiler_params=pltpu.CompilerParams(use_tc_tiling_on_sc=False),
    scratch_types=[
        pltpu.VMEM_SHARED((SUMROWS, D), jnp.float32),
        pltpu.VMEM((SEG_CH, CW), jnp.int32),
        pltpu.VMEM((SEG_CH, CW), jnp.int32),
        pltpu.VMEM((RB, CW, D), jnp.float32),
        pltpu.SemaphoreType.DMA((RB,)),
        pltpu.SemaphoreType.DMA((RB,)),
    ],
)
def _sc_segsum(msgs, zrows, tgt_all, pos_all, out, sums, idx_v, pos_v, ring,
               semL, semA):
    c = lax.axis_index("c")
    s = lax.axis_index("s")
    pltpu.sync_copy(tgt_all.at[c, s], idx_v)
    pltpu.sync_copy(pos_all.at[c, s], pos_v)
    base = s * SEG_ROWS

    pltpu.sync_copy(zrows, ring.at[0])

    def zfire(kk, _):
        pltpu.async_copy(ring.at[0], sums.at[idx_v.at[kk]], semA.at[0])
        return 0

    lax.fori_loop(0, SEG_CH, zfire, 0)

    def zdrain(kk, _):
        pltpu.make_async_copy(ring.at[0], sums.at[idx_v.at[0]],
                              semA.at[0]).wait()
        return 0

    lax.fori_loop(0, SEG_CH, zdrain, 0)
    plsc.subcore_barrier()

    def agroup(g, _):
        for b in range(RB):
            k = g * RB + b

            @pl.when(g > 0)
            def _wait_add():
                pltpu.make_async_copy(ring.at[b], sums.at[idx_v.at[0]],
                                      semA.at[b]).wait()

            pltpu.async_copy(msgs.at[pl.ds(base + k * CW, CW)], ring.at[b],
                             semL.at[b])
        for b in range(RB):
            k = g * RB + b
            pltpu.make_async_copy(msgs.at[pl.ds(base, CW)], ring.at[b],
                                  semL.at[b]).wait()
            pltpu.async_copy(ring.at[b], sums.at[idx_v.at[k]], semA.at[b],
                             add=True)
        return 0

    lax.fori_loop(0, NG, agroup, 0)
    for b in range(RB):
        pltpu.make_async_copy(ring.at[b], sums.at[idx_v.at[0]],
                              semA.at[b]).wait()
    plsc.subcore_barrier()

    def bgroup(g, _):
        for b in range(RB):
            k = g * RB + b

            @pl.when(g > 0)
            def _wait_out():
                pltpu.make_async_copy(ring.at[b], out.at[pos_v.at[0]],
                                      semA.at[b]).wait()

            pltpu.async_copy(sums.at[idx_v.at[k]], ring.at[b], semL.at[b])
        for b in range(RB):
            k = g * RB + b
            pltpu.make_async_copy(sums.at[idx_v.at[0]], ring.at[b],
                                  semL.at[b]).wait()
            pltpu.async_copy(ring.at[b], out.at[pos_v.at[k]], semA.at[b])
        return 0

    lax.fori_loop(0, NG, bgroup, 0)
    for b in range(RB):
        pltpu.make_async_copy(ring.at[b], out.at[pos_v.at[0]],
                              semA.at[b]).wait()


@functools.partial(
    pl.kernel,
    out_type=jax.ShapeDtypeStruct((XROWS, D), jnp.float32),
    mesh=_MESH,
    compiler_params=pltpu.CompilerParams(use_tc_tiling_on_sc=False),
    scratch_types=[
        pltpu.VMEM((HALF // NSUB, D), jnp.float32),
        pltpu.VMEM((SEG_CH, CW), jnp.int32),
        pltpu.VMEM((RB, CW, D), jnp.float32),
        pltpu.SemaphoreType.DMA((RB,)),
        pltpu.SemaphoreType.DMA((RB,)),
    ],
)
def _sc_buildx(emb, upd, xtgt_all, x, cbuf, idx_v, ring, semL, semA):
    c = lax.axis_index("c")
    s = lax.axis_index("s")
    tbase = c * HALF + s * (HALF // NSUB)
    pltpu.sync_copy(emb.at[pl.ds(tbase, HALF // NSUB)], cbuf)
    pltpu.sync_copy(cbuf, x.at[pl.ds(tbase, HALF // NSUB)])
    plsc.subcore_barrier()

    pltpu.sync_copy(xtgt_all.at[c, s], idx_v)
    base = s * SEG_ROWS

    def group(g, _):
        for b in range(RB):
            k = g * RB + b

            @pl.when(g > 0)
            def _wait_scat():
                pltpu.make_async_copy(ring.at[b], x.at[idx_v.at[0]],
                                      semA.at[b]).wait()

            pltpu.async_copy(upd.at[pl.ds(base + k * CW, CW)], ring.at[b],
                             semL.at[b])
        for b in range(RB):
            k = g * RB + b
            pltpu.make_async_copy(upd.at[pl.ds(base, CW)], ring.at[b],
                                  semL.at[b]).wait()
            pltpu.async_copy(ring.at[b], x.at[idx_v.at[k]], semA.at[b])
        return 0

    lax.fori_loop(0, NG, group, 0)
    for b in range(RB):
        pltpu.make_async_copy(ring.at[b], x.at[idx_v.at[0]], semA.at[b]).wait()


def _spec_a():
    return pl.BlockSpec((BLK, D), lambda i: (i, 0))


def _spec_b():
    return pl.BlockSpec((BLK, D), lambda i: ((i + HB) % NB, 0))


def _wspec(shape):
    nd = len(shape)
    return pl.BlockSpec(shape, lambda i: (0,) * nd)


def _tc_msgs_body(xa, xb, w1, b1, w2, b2, o):
    a = jnp.concatenate([xa[...], xb[...]], axis=1)
    h = jnp.maximum(jnp.dot(a, w1[...], preferred_element_type=jnp.float32)
                    + b1[...], 0.0)
    o[...] = jnp.dot(h, w2[...], preferred_element_type=jnp.float32) + b2[...]


def _tc_msgs(rows, W1, b1, W2, b2, swapped):
    first, second = (_spec_b(), _spec_a()) if swapped else (_spec_a(), _spec_b())
    return pl.pallas_call(
        _tc_msgs_body,
        grid=(NB,),
        in_specs=[first, second, _wspec((2 * D, H)), _wspec((1, H)),
                  _wspec((H, D)), _wspec((1, D))],
        out_specs=_spec_a(),
        out_shape=jax.ShapeDtypeStruct((TOT, D), jnp.float32),
    )(rows, rows, W1, b1, W2, b2)


def _tc_upd_body(xt, sv, cv, wu, bu, o):
    mean = sv[...] / jnp.maximum(cv[...], 1.0)
    a = jnp.concatenate([xt[...], mean], axis=1)
    o[...] = jnp.maximum(
        jnp.dot(a, wu[...], preferred_element_type=jnp.float32) + bu[...], 0.0)


def _tc_upd(rows, sv, cv, Wu, bu, swapped):
    xt_spec = _spec_a() if swapped else _spec_b()
    return pl.pallas_call(
        _tc_upd_body,
        grid=(NB,),
        in_specs=[xt_spec, _spec_a(), _spec_a(), _wspec((2 * D, D)),
                  _wspec((1, D))],
        out_specs=_spec_a(),
        out_shape=jax.ShapeDtypeStruct((TOT, D), jnp.float32),
    )(rows, sv, cv, Wu, bu)


def _tc_pred_body(he, ae, w1, b1, w2, b2, w3, b3, o):
    a = jnp.concatenate([he[...], ae[...]], axis=1)
    p = jnp.maximum(jnp.dot(a, w1[...], preferred_element_type=jnp.float32)
                    + b1[...], 0.0)
    p = jnp.maximum(jnp.dot(p, w2[...], preferred_element_type=jnp.float32)
                    + b2[...], 0.0)
    z = jnp.dot(p, w3[...], preferred_element_type=jnp.float32) + b3[...]
    o[...] = 1.0 / (1.0 + jnp.exp(-z))


def _tc_pred(rows_ha, Wp1, bp1, Wp2, bp2, Wp3, bp3):
    nb = B // BLK
    he_spec = pl.BlockSpec((BLK, D), lambda i: (i, 0))
    ae_spec = pl.BlockSpec((BLK, D), lambda i: (i + nb, 0))
    return pl.pallas_call(
        _tc_pred_body,
        grid=(nb,),
        in_specs=[he_spec, ae_spec, _wspec((2 * D, H)), _wspec((1, H)),
                  _wspec((H, D)), _wspec((1, D)), _wspec((D, 1)),
                  _wspec((1, 1))],
        out_specs=pl.BlockSpec((BLK, 1), lambda i: (i, 0)),
        out_shape=jax.ShapeDtypeStruct((B, 1), jnp.float32),
    )(rows_ha, rows_ha, Wp1, bp1, Wp2, bp2, Wp3, bp3)


def kernel(emb, W1, b1, W2, b2, Wu, bu, Wp1, bp1, Wp2, bp2, Wp3, bp3,
           home_ids, away_ids, adjacency):
    adjacency = adjacency.astype(jnp.int32)
    home_ids = home_ids.astype(jnp.int32)
    away_ids = away_ids.astype(jnp.int32)
    src = adjacency[:, 0]
    dst = adjacency[:, 1]

    pad0 = (jnp.arange(EP - E, dtype=jnp.int32) * 523) % N_TEAMS
    padm = jnp.full((EP - E,), -1, jnp.int32)
    idxcat = jnp.concatenate([src, pad0, dst, pad0])
    tgt = jnp.concatenate([dst, padm, src, padm])

    in0 = (tgt >= 0) & (tgt < HALF)
    in1 = tgt >= HALF
    pos = jnp.arange(TOT, dtype=jnp.int32)
    slop = pos & (NSLOP - 1)
    tgt_all = jnp.stack([jnp.where(in0, tgt, DUMP + slop),
                         jnp.where(in1, tgt - HALF, DUMP + slop)])
    tgt_all = tgt_all.reshape(2, NSUB, SEG_CH, CW)
    pos_all = jnp.stack([jnp.where(in0, pos, OUT_DUMP + slop),
                         jnp.where(in1, pos, OUT_DUMP + slop)])
    pos_all = pos_all.reshape(2, NSUB, SEG_CH, CW)
    xtgt_all = jnp.stack([jnp.where(in0, tgt, XDUMP + slop),
                          jnp.where(in1, tgt, XDUMP + slop)])
    xtgt_all = xtgt_all.reshape(2, NSUB, SEG_CH, CW)
    idx3 = idxcat.reshape(NW, NCH, CW)
    ha3 = jnp.concatenate([home_ids, away_ids]).reshape(NW, 2 * B // NW // CW, CW)

    zrows = jnp.zeros((CW, D), jnp.float32)
    ones = jnp.ones((TOT, D), jnp.float32)
    b1r = b1.reshape(1, H)
    b2r = b2.reshape(1, D)
    bur = bu.reshape(1, D)
    bp1r = bp1.reshape(1, H)
    bp2r = bp2.reshape(1, D)
    bp3r = bp3.reshape(1, 1)

    rows = _sc_gather_edges(emb, idx3)
    cv = _sc_segsum(ones, zrows, tgt_all, pos_all)[:TOT]

    swapped = False
    for _ in range(PASSES):
        msgs = _tc_msgs(rows, W1, b1r, W2, b2r, swapped)
        sv = _sc_segsum(msgs, zrows, tgt_all, pos_all)[:TOT]
        rows = _tc_upd(rows, sv, cv, Wu, bur, swapped)
        swapped = True

    x = _sc_buildx(emb, rows, xtgt_all)
    rows_ha = _sc_gather_pairs(x, ha3)
    return _tc_pred(rows_ha, Wp1, bp1r, Wp2, bp2r, Wp3, bp3r)


# speedup vs baseline: 4.2762x; 1.0859x over previous
import functools

import jax
import jax.numpy as jnp
from jax import lax
from jax.experimental import pallas as pl
from jax.experimental.pallas import tpu as pltpu
from jax.experimental.pallas import tpu_sc as plsc

N_TEAMS = 100000
D = 32
H = 64
E = 20000
B = 16384
PASSES = 3

EP = 20480
TOT = 2 * EP
NW = 32
NSUB = 16
HALF = 50000
NSLOP = 64
DUMP = HALF
SUMROWS = HALF + NSLOP
OUT_DUMP = TOT
XDUMP = N_TEAMS
XROWS = N_TEAMS + NSLOP
CW = 128
NCH = TOT // NW // CW
CHUNK = NCH * CW
SEG_CH = TOT // NSUB // CW
SEG_ROWS = SEG_CH * CW
RB = 5
NG = SEG_CH // RB

_MESH = plsc.VectorSubcoreMesh(core_axis_name="c", subcore_axis_name="s")

BLK = 2048
NB = TOT // BLK
HB = EP // BLK


def _make_sc_gather(nch):
    rows_per_tile = nch * CW

    @functools.partial(
        pl.kernel,
        out_type=jax.ShapeDtypeStruct((NW * rows_per_tile, D), jnp.float32),
        mesh=_MESH,
        compiler_params=pltpu.CompilerParams(use_tc_tiling_on_sc=False),
        scratch_types=[
            pltpu.VMEM((nch, CW), jnp.int32),
            pltpu.VMEM((rows_per_tile, D), jnp.float32),
            pltpu.SemaphoreType.DMA,
        ],
    )
    def k(table, idx3, out, idx_v, buf, sem):
        c = lax.axis_index("c")
        s = lax.axis_index("s")
        w = c * NSUB + s
        pltpu.sync_copy(idx3.at[w], idx_v)

        def fire(kk, _):
            pltpu.async_copy(table.at[idx_v.at[kk]],
                             buf.at[pl.ds(kk * CW, CW)], sem)
            return 0

        lax.fori_loop(0, nch, fire, 0)

        def drain(kk, _):
            pltpu.make_async_copy(table.at[idx_v.at[0]],
                                  buf.at[pl.ds(0, CW)], sem).wait()
            return 0

        lax.fori_loop(0, nch, drain, 0)
        pltpu.sync_copy(buf, out.at[pl.ds(w * rows_per_tile, rows_per_tile)])

    return k


_sc_gather_edges = _make_sc_gather(NCH)
_sc_gather_pairs = _make_sc_gather(2 * B // NW // CW)


@functools.partial(
    pl.kernel,
    out_type=jax.ShapeDtypeStruct((TOT + NSLOP, D), jnp.float32),
    mesh=_MESH,
    compiler_params=pltpu.CompilerParams(use_tc_tiling_on_sc=False),
    scratch_types=[
        pltpu.VMEM_SHARED((SUMROWS, D), jnp.float32),
        pltpu.VMEM((SEG_CH, CW), jnp.int32),
        pltpu.VMEM((SEG_CH, CW), jnp.int32),
        pltpu.VMEM((RB, CW, D), jnp.float32),
        pltpu.SemaphoreType.DMA((RB,)),
        pltpu.SemaphoreType.DMA((RB,)),
    ],
)
def _sc_segsum(msgs, zrows, tgt_all, pos_all, out, sums, idx_v, pos_v, ring,
               semL, semA):
    c = lax.axis_index("c")
    s = lax.axis_index("s")
    pltpu.sync_copy(tgt_all.at[c, s], idx_v)
    pltpu.sync_copy(pos_all.at[c, s], pos_v)
    base = s * SEG_ROWS

    pltpu.sync_copy(zrows, ring.at[0])

    def zfire(kk, _):
        pltpu.async_copy(ring.at[0], sums.at[idx_v.at[kk]], semA.at[0])
        return 0

    lax.fori_loop(0, SEG_CH, zfire, 0)

    def zdrain(kk, _):
        pltpu.make_async_copy(ring.at[0], sums.at[idx_v.at[0]],
                              semA.at[0]).wait()
        return 0

    lax.fori_loop(0, SEG_CH, zdrain, 0)
    plsc.subcore_barrier()

    def agroup(g, _):
        for b in range(RB):
            k = g * RB + b

            @pl.when(g > 0)
            def _wait_add():
                pltpu.make_async_copy(ring.at[b], sums.at[idx_v.at[0]],
                                      semA.at[b]).wait()

            pltpu.async_copy(msgs.at[pl.ds(base + k * CW, CW)], ring.at[b],
                             semL.at[b])
        for b in range(RB):
            k = g * RB + b
            pltpu.make_async_copy(msgs.at[pl.ds(base, CW)], ring.at[b],
                                  semL.at[b]).wait()
            pltpu.async_copy(ring.at[b], sums.at[idx_v.at[k]], semA.at[b],
                             add=True)
        return 0

    lax.fori_loop(0, NG, agroup, 0)
    for b in range(RB):
        pltpu.make_async_copy(ring.at[b], sums.at[idx_v.at[0]],
                              semA.at[b]).wait()
    plsc.subcore_barrier()

    def bgroup(g, _):
        for b in range(RB):
            k = g * RB + b

            @pl.when(g > 0)
            def _wait_out():
                pltpu.make_async_copy(ring.at[b], out.at[pos_v.at[0]],
                                      semA.at[b]).wait()

            pltpu.async_copy(sums.at[idx_v.at[k]], ring.at[b], semL.at[b])
        for b in range(RB):
            k = g * RB + b
            pltpu.make_async_copy(sums.at[idx_v.at[0]], ring.at[b],
                                  semL.at[b]).wait()
            pltpu.async_copy(ring.at[b], out.at[pos_v.at[k]], semA.at[b])
        return 0

    lax.fori_loop(0, NG, bgroup, 0)
    for b in range(RB):
        pltpu.make_async_copy(ring.at[b], out.at[pos_v.at[0]],
                              semA.at[b]).wait()


CVW = 16


@functools.partial(
    pl.kernel,
    out_type=jax.ShapeDtypeStruct((TOT + NSLOP, CVW), jnp.float32),
    mesh=_MESH,
    compiler_params=pltpu.CompilerParams(use_tc_tiling_on_sc=False),
    scratch_types=[
        pltpu.VMEM_SHARED((SUMROWS, CVW), jnp.float32),
        pltpu.VMEM((SEG_CH, CW), jnp.int32),
        pltpu.VMEM((SEG_CH, CW), jnp.int32),
        pltpu.VMEM((RB, CW, CVW), jnp.float32),
        pltpu.SemaphoreType.DMA((RB,)),
        pltpu.SemaphoreType.DMA((RB,)),
    ],
)
def _sc_cnt(zrows, orows, tgt_all, pos_all, out, cnts, idx_v, pos_v, ring,
            semL, semA):
    c = lax.axis_index("c")
    s = lax.axis_index("s")
    pltpu.sync_copy(tgt_all.at[c, s], idx_v)
    pltpu.sync_copy(pos_all.at[c, s], pos_v)
    pltpu.sync_copy(zrows, ring.at[0])
    pltpu.sync_copy(orows, ring.at[1])

    def zfire(kk, _):
        pltpu.async_copy(ring.at[0], cnts.at[idx_v.at[kk]], semA.at[0])
        return 0

    lax.fori_loop(0, SEG_CH, zfire, 0)

    def zdrain(kk, _):
        pltpu.make_async_copy(ring.at[0], cnts.at[idx_v.at[0]],
                              semA.at[0]).wait()
        return 0

    lax.fori_loop(0, SEG_CH, zdrain, 0)
    plsc.subcore_barrier()

    def afire(kk, _):
        pltpu.async_copy(ring.at[1], cnts.at[idx_v.at[kk]], semA.at[1],
                         add=True)
        return 0

    lax.fori_loop(0, SEG_CH, afire, 0)

    def adrain(kk, _):
        pltpu.make_async_copy(ring.at[1], cnts.at[idx_v.at[0]],
                              semA.at[1]).wait()
        return 0

    lax.fori_loop(0, SEG_CH, adrain, 0)
    plsc.subcore_barrier()

    def bgroup(g, _):
        for b in range(RB):
            k = g * RB + b

            @pl.when(g > 0)
            def _wait_out():
                pltpu.make_async_copy(ring.at[b], out.at[pos_v.at[0]],
                                      semA.at[b]).wait()

            pltpu.async_copy(cnts.at[idx_v.at[k]], ring.at[b], semL.at[b])
        for b in range(RB):
            k = g * RB + b
            pltpu.make_async_copy(cnts.at[idx_v.at[0]], ring.at[b],
                                  semL.at[b]).wait()
            pltpu.async_copy(ring.at[b], out.at[pos_v.at[k]], semA.at[b])
        return 0

    lax.fori_loop(0, NG, bgroup, 0)
    for b in range(RB):
        pltpu.make_async_copy(ring.at[b], out.at[pos_v.at[0]],
                              semA.at[b]).wait()


@functools.partial(
    pl.kernel,
    out_type=jax.ShapeDtypeStruct((XROWS, D), jnp.float32),
    mesh=_MESH,
    compiler_params=pltpu.CompilerParams(use_tc_tiling_on_sc=False),
    scratch_types=[
        pltpu.VMEM((HALF // NSUB, D), jnp.float32),
        pltpu.VMEM((SEG_CH, CW), jnp.int32),
        pltpu.VMEM((RB, CW, D), jnp.float32),
        pltpu.SemaphoreType.DMA((RB,)),
        pltpu.SemaphoreType.DMA((RB,)),
    ],
)
def _sc_buildx(emb, upd, xtgt_all, x, cbuf, idx_v, ring, semL, semA):
    c = lax.axis_index("c")
    s = lax.axis_index("s")
    tbase = c * HALF + s * (HALF // NSUB)
    pltpu.sync_copy(emb.at[pl.ds(tbase, HALF // NSUB)], cbuf)
    pltpu.sync_copy(cbuf, x.at[pl.ds(tbase, HALF // NSUB)])
    plsc.subcore_barrier()

    pltpu.sync_copy(xtgt_all.at[c, s], idx_v)
    base = s * SEG_ROWS

    def group(g, _):
        for b in range(RB):
            k = g * RB + b

            @pl.when(g > 0)
            def _wait_scat():
                pltpu.make_async_copy(ring.at[b], x.at[idx_v.at[0]],
                                      semA.at[b]).wait()

            pltpu.async_copy(upd.at[pl.ds(base + k * CW, CW)], ring.at[b],
                             semL.at[b])
        for b in range(RB):
            k = g * RB + b
            pltpu.make_async_copy(upd.at[pl.ds(base, CW)], ring.at[b],
                                  semL.at[b]).wait()
            pltpu.async_copy(ring.at[b], x.at[idx_v.at[k]], semA.at[b])
        return 0

    lax.fori_loop(0, NG, group, 0)
    for b in range(RB):
        pltpu.make_async_copy(ring.at[b], x.at[idx_v.at[0]], semA.at[b]).wait()


def _spec_a():
    return pl.BlockSpec((BLK, D), lambda i: (i, 0))


def _spec_b():
    return pl.BlockSpec((BLK, D), lambda i: ((i + HB) % NB, 0))


def _wspec(shape):
    nd = len(shape)
    return pl.BlockSpec(shape, lambda i: (0,) * nd)


def _tc_msgs_body(xa, xb, w1, b1, w2, b2, o):
    a = jnp.concatenate([xa[...], xb[...]], axis=1)
    h = jnp.maximum(jnp.dot(a, w1[...], preferred_element_type=jnp.float32)
                    + b1[...], 0.0)
    o[...] = jnp.dot(h, w2[...], preferred_element_type=jnp.float32) + b2[...]


def _tc_msgs(rows, W1, b1, W2, b2, swapped):
    first, second = (_spec_b(), _spec_a()) if swapped else (_spec_a(), _spec_b())
    return pl.pallas_call(
        _tc_msgs_body,
        grid=(NB,),
        in_specs=[first, second, _wspec((2 * D, H)), _wspec((1, H)),
                  _wspec((H, D)), _wspec((1, D))],
        out_specs=_spec_a(),
        out_shape=jax.ShapeDtypeStruct((TOT, D), jnp.float32),
    )(rows, rows, W1, b1, W2, b2)


def _tc_upd_body(xt, sv, cv, wu, bu, o):
    mean = sv[...] / jnp.maximum(cv[...][:, 0:1], 1.0)
    a = jnp.concatenate([xt[...], mean], axis=1)
    o[...] = jnp.maximum(
        jnp.dot(a, wu[...], preferred_element_type=jnp.float32) + bu[...], 0.0)


def _tc_upd(rows, sv, cv, Wu, bu, swapped):
    xt_spec = _spec_a() if swapped else _spec_b()
    return pl.pallas_call(
        _tc_upd_body,
        grid=(NB,),
        in_specs=[xt_spec, _spec_a(),
                  pl.BlockSpec((BLK, CVW), lambda i: (i, 0)),
                  _wspec((2 * D, D)), _wspec((1, D))],
        out_specs=_spec_a(),
        out_shape=jax.ShapeDtypeStruct((TOT, D), jnp.float32),
    )(rows, sv, cv, Wu, bu)


def _tc_pred_body(he, ae, w1, b1, w2, b2, w3, b3, o):
    a = jnp.concatenate([he[...], ae[...]], axis=1)
    p = jnp.maximum(jnp.dot(a, w1[...], preferred_element_type=jnp.float32)
                    + b1[...], 0.0)
    p = jnp.maximum(jnp.dot(p, w2[...], preferred_element_type=jnp.float32)
                    + b2[...], 0.0)
    z = jnp.dot(p, w3[...], preferred_element_type=jnp.float32) + b3[...]
    o[...] = 1.0 / (1.0 + jnp.exp(-z))


def _tc_pred(rows_ha, Wp1, bp1, Wp2, bp2, Wp3, bp3):
    nb = B // BLK
    he_spec = pl.BlockSpec((BLK, D), lambda i: (i, 0))
    ae_spec = pl.BlockSpec((BLK, D), lambda i: (i + nb, 0))
    return pl.pallas_call(
        _tc_pred_body,
        grid=(nb,),
        in_specs=[he_spec, ae_spec, _wspec((2 * D, H)), _wspec((1, H)),
                  _wspec((H, D)), _wspec((1, D)), _wspec((D, 1)),
                  _wspec((1, 1))],
        out_specs=pl.BlockSpec((BLK, 1), lambda i: (i, 0)),
        out_shape=jax.ShapeDtypeStruct((B, 1), jnp.float32),
    )(rows_ha, rows_ha, Wp1, bp1, Wp2, bp2, Wp3, bp3)


def kernel(emb, W1, b1, W2, b2, Wu, bu, Wp1, bp1, Wp2, bp2, Wp3, bp3,
           home_ids, away_ids, adjacency):
    adjacency = adjacency.astype(jnp.int32)
    home_ids = home_ids.astype(jnp.int32)
    away_ids = away_ids.astype(jnp.int32)
    src = adjacency[:, 0]
    dst = adjacency[:, 1]

    pad0 = (jnp.arange(EP - E, dtype=jnp.int32) * 523) % N_TEAMS
    padm = jnp.full((EP - E,), -1, jnp.int32)
    idxcat = jnp.concatenate([src, pad0, dst, pad0])
    tgt = jnp.concatenate([dst, padm, src, padm])

    in0 = (tgt >= 0) & (tgt < HALF)
    in1 = tgt >= HALF
    pos = jnp.arange(TOT, dtype=jnp.int32)
    slop = pos & (NSLOP - 1)
    tgt_all = jnp.stack([jnp.where(in0, tgt, DUMP + slop),
                         jnp.where(in1, tgt - HALF, DUMP + slop)])
    tgt_all = tgt_all.reshape(2, NSUB, SEG_CH, CW)
    pos_all = jnp.stack([jnp.where(in0, pos, OUT_DUMP + slop),
                         jnp.where(in1, pos, OUT_DUMP + slop)])
    pos_all = pos_all.reshape(2, NSUB, SEG_CH, CW)
    xtgt_all = jnp.stack([jnp.where(in0, tgt, XDUMP + slop),
                          jnp.where(in1, tgt, XDUMP + slop)])
    xtgt_all = xtgt_all.reshape(2, NSUB, SEG_CH, CW)
    idx3 = idxcat.reshape(NW, NCH, CW)
    ha3 = jnp.concatenate([home_ids, away_ids]).reshape(NW, 2 * B // NW // CW, CW)

    zrows = jnp.zeros((CW, D), jnp.float32)
    zrows16 = jnp.zeros((CW, CVW), jnp.float32)
    orows16 = jnp.ones((CW, CVW), jnp.float32)
    b1r = b1.reshape(1, H)
    b2r = b2.reshape(1, D)
    bur = bu.reshape(1, D)
    bp1r = bp1.reshape(1, H)
    bp2r = bp2.reshape(1, D)
    bp3r = bp3.reshape(1, 1)

    rows = _sc_gather_edges(emb, idx3)
    cv = _sc_cnt(zrows16, orows16, tgt_all, pos_all)

    swapped = False
    for _ in range(PASSES):
        msgs = _tc_msgs(rows, W1, b1r, W2, b2r, swapped)
        sv = _sc_segsum(msgs, zrows, tgt_all, pos_all)
        rows = _tc_upd(rows, sv, cv, Wu, bur, swapped)
        swapped = True

    x = _sc_buildx(emb, rows, xtgt_all)
    rows_ha = _sc_gather_pairs(x, ha3)
    return _tc_pred(rows_ha, Wp1, bp1r, Wp2, bp2r, Wp3, bp3r)


# speedup vs baseline: 4.2995x; 1.0055x over previous
import functools

import jax
import jax.numpy as jnp
from jax import lax
from jax.experimental import pallas as pl
from jax.experimental.pallas import tpu as pltpu
from jax.experimental.pallas import tpu_sc as plsc

N_TEAMS = 100000
D = 32
H = 64
E = 20000
B = 16384
PASSES = 3

EP = 20480
TOT = 2 * EP
NW = 32
NSUB = 16
HALF = 50000
NSLOP = 64
DUMP = HALF
SUMROWS = HALF + NSLOP
OUT_DUMP = TOT
XDUMP = N_TEAMS
XROWS = N_TEAMS + NSLOP
CW = 128
NCH = TOT // NW // CW
CHUNK = NCH * CW
SEG_CH = TOT // NSUB // CW
SEG_ROWS = SEG_CH * CW
RB = 5
NG = SEG_CH // RB

_MESH = plsc.VectorSubcoreMesh(core_axis_name="c", subcore_axis_name="s")

BLK = 2048
NB = TOT // BLK
HB = EP // BLK


def _make_sc_gather(nch):
    rows_per_tile = nch * CW

    @functools.partial(
        pl.kernel,
        out_type=jax.ShapeDtypeStruct((NW * rows_per_tile, D), jnp.float32),
        mesh=_MESH,
        compiler_params=pltpu.CompilerParams(use_tc_tiling_on_sc=False),
        scratch_types=[
            pltpu.VMEM((nch, CW), jnp.int32),
            pltpu.VMEM((rows_per_tile, D), jnp.float32),
            pltpu.SemaphoreType.DMA,
        ],
    )
    def k(table, idx3, out, idx_v, buf, sem):
        c = lax.axis_index("c")
        s = lax.axis_index("s")
        w = c * NSUB + s
        pltpu.sync_copy(idx3.at[w], idx_v)

        def fire(kk, _):
            pltpu.async_copy(table.at[idx_v.at[kk]],
                             buf.at[pl.ds(kk * CW, CW)], sem)
            return 0

        lax.fori_loop(0, nch, fire, 0)

        def drain(kk, _):
            pltpu.make_async_copy(table.at[idx_v.at[0]],
                                  buf.at[pl.ds(0, CW)], sem).wait()
            return 0

        lax.fori_loop(0, nch, drain, 0)
        pltpu.sync_copy(buf, out.at[pl.ds(w * rows_per_tile, rows_per_tile)])

    return k


_sc_gather_edges = _make_sc_gather(NCH)
_sc_gather_pairs = _make_sc_gather(2 * B // NW // CW)


@functools.partial(
    pl.kernel,
    out_type=jax.ShapeDtypeStruct((TOT + NSLOP, D), jnp.float32),
    mesh=_MESH,
    compiler_params=pltpu.CompilerParams(use_tc_tiling_on_sc=False),
    scratch_types=[
        pltpu.VMEM_SHARED((SUMROWS, D), jnp.float32),
        pltpu.VMEM((SEG_CH, CW), jnp.int32),
        pltpu.VMEM((SEG_CH, CW), jnp.int32),
        pltpu.VMEM((RB + 1, CW, D), jnp.float32),
        pltpu.SemaphoreType.DMA((RB,)),
        pltpu.SemaphoreType.DMA((RB,)),
    ],
)
def _sc_segsum(msgs, zrows, tgt_all, pos_all, out, sums, idx_v, pos_v, ring,
               semL, semA):
    c = lax.axis_index("c")
    s = lax.axis_index("s")
    pltpu.sync_copy(tgt_all.at[c, s], idx_v)
    pltpu.sync_copy(pos_all.at[c, s], pos_v)
    base = s * SEG_ROWS

    pltpu.sync_copy(zrows, ring.at[RB])
    for b in range(RB):
        pltpu.async_copy(msgs.at[pl.ds(base + b * CW, CW)], ring.at[b],
                         semL.at[b])

    def zfire(kk, _):
        pltpu.async_copy(ring.at[RB], sums.at[idx_v.at[kk]], semA.at[0])
        return 0

    lax.fori_loop(0, SEG_CH, zfire, 0)

    def zdrain(kk, _):
        pltpu.make_async_copy(ring.at[RB], sums.at[idx_v.at[0]],
                              semA.at[0]).wait()
        return 0

    lax.fori_loop(0, SEG_CH, zdrain, 0)
    plsc.subcore_barrier()

    def agroup(g, _):
        for b in range(RB):
            k = g * RB + b
            pltpu.make_async_copy(msgs.at[pl.ds(base, CW)], ring.at[b],
                                  semL.at[b]).wait()
            pltpu.async_copy(ring.at[b], sums.at[idx_v.at[k]], semA.at[b],
                             add=True)
        for b in range(RB):
            k = g * RB + b

            @pl.when(g < NG - 1)
            def _prefire_next():
                pltpu.make_async_copy(ring.at[b], sums.at[idx_v.at[0]],
                                      semA.at[b]).wait()
                pltpu.async_copy(msgs.at[pl.ds(base + (k + RB) * CW, CW)],
                                 ring.at[b], semL.at[b])

        return 0

    lax.fori_loop(0, NG, agroup, 0)
    for b in range(RB):
        pltpu.make_async_copy(ring.at[b], sums.at[idx_v.at[0]],
                              semA.at[b]).wait()
    plsc.subcore_barrier()

    def bgroup(g, _):
        for b in range(RB):
            k = g * RB + b

            @pl.when(g > 0)
            def _wait_out():
                pltpu.make_async_copy(ring.at[b], out.at[pos_v.at[0]],
                                      semA.at[b]).wait()

            pltpu.async_copy(sums.at[idx_v.at[k]], ring.at[b], semL.at[b])
        for b in range(RB):
            k = g * RB + b
            pltpu.make_async_copy(sums.at[idx_v.at[0]], ring.at[b],
                                  semL.at[b]).wait()
            pltpu.async_copy(ring.at[b], out.at[pos_v.at[k]], semA.at[b])
        return 0

    lax.fori_loop(0, NG, bgroup, 0)
    for b in range(RB):
        pltpu.make_async_copy(ring.at[b], out.at[pos_v.at[0]],
                              semA.at[b]).wait()


CVW = 16


@functools.partial(
    pl.kernel,
    out_type=jax.ShapeDtypeStruct((TOT + NSLOP, CVW), jnp.float32),
    mesh=_MESH,
    compiler_params=pltpu.CompilerParams(use_tc_tiling_on_sc=False),
    scratch_types=[
        pltpu.VMEM_SHARED((SUMROWS, CVW), jnp.float32),
        pltpu.VMEM((SEG_CH, CW), jnp.int32),
        pltpu.VMEM((SEG_CH, CW), jnp.int32),
        pltpu.VMEM((RB, CW, CVW), jnp.float32),
        pltpu.SemaphoreType.DMA((RB,)),
        pltpu.SemaphoreType.DMA((RB,)),
    ],
)
def _sc_cnt(zrows, orows, tgt_all, pos_all, out, cnts, idx_v, pos_v, ring,
            semL, semA):
    c = lax.axis_index("c")
    s = lax.axis_index("s")
    pltpu.sync_copy(tgt_all.at[c, s], idx_v)
    pltpu.sync_copy(pos_all.at[c, s], pos_v)
    pltpu.sync_copy(zrows, ring.at[0])
    pltpu.sync_copy(orows, ring.at[1])

    def zfire(kk, _):
        pltpu.async_copy(ring.at[0], cnts.at[idx_v.at[kk]], semA.at[0])
        return 0

    lax.fori_loop(0, SEG_CH, zfire, 0)

    def zdrain(kk, _):
        pltpu.make_async_copy(ring.at[0], cnts.at[idx_v.at[0]],
                              semA.at[0]).wait()
        return 0

    lax.fori_loop(0, SEG_CH, zdrain, 0)
    plsc.subcore_barrier()

    def afire(kk, _):
        pltpu.async_copy(ring.at[1], cnts.at[idx_v.at[kk]], semA.at[1],
                         add=True)
        return 0

    lax.fori_loop(0, SEG_CH, afire, 0)

    def adrain(kk, _):
        pltpu.make_async_copy(ring.at[1], cnts.at[idx_v.at[0]],
                              semA.at[1]).wait()
        return 0

    lax.fori_loop(0, SEG_CH, adrain, 0)
    plsc.subcore_barrier()

    def bgroup(g, _):
        for b in range(RB):
            k = g * RB + b

            @pl.when(g > 0)
            def _wait_out():
                pltpu.make_async_copy(ring.at[b], out.at[pos_v.at[0]],
                                      semA.at[b]).wait()

            pltpu.async_copy(cnts.at[idx_v.at[k]], ring.at[b], semL.at[b])
        for b in range(RB):
            k = g * RB + b
            pltpu.make_async_copy(cnts.at[idx_v.at[0]], ring.at[b],
                                  semL.at[b]).wait()
            pltpu.async_copy(ring.at[b], out.at[pos_v.at[k]], semA.at[b])
        return 0

    lax.fori_loop(0, NG, bgroup, 0)
    for b in range(RB):
        pltpu.make_async_copy(ring.at[b], out.at[pos_v.at[0]],
                              semA.at[b]).wait()


@functools.partial(
    pl.kernel,
    out_type=jax.ShapeDtypeStruct((XROWS, D), jnp.float32),
    mesh=_MESH,
    compiler_params=pltpu.CompilerParams(use_tc_tiling_on_sc=False),
    scratch_types=[
        pltpu.VMEM((HALF // NSUB, D), jnp.float32),
        pltpu.VMEM((SEG_CH, CW), jnp.int32),
        pltpu.VMEM((RB, CW, D), jnp.float32),
        pltpu.SemaphoreType.DMA((RB,)),
        pltpu.SemaphoreType.DMA((RB,)),
    ],
)
def _sc_buildx(emb, upd, xtgt_all, x, cbuf, idx_v, ring, semL, semA):
    c = lax.axis_index("c")
    s = lax.axis_index("s")
    tbase = c * HALF + s * (HALF // NSUB)
    pltpu.sync_copy(emb.at[pl.ds(tbase, HALF // NSUB)], cbuf)
    pltpu.sync_copy(cbuf, x.at[pl.ds(tbase, HALF // NSUB)])
    plsc.subcore_barrier()

    pltpu.sync_copy(xtgt_all.at[c, s], idx_v)
    base = s * SEG_ROWS

    def group(g, _):
        for b in range(RB):
            k = g * RB + b

            @pl.when(g > 0)
            def _wait_scat():
                pltpu.make_async_copy(ring.at[b], x.at[idx_v.at[0]],
                                      semA.at[b]).wait()

            pltpu.async_copy(upd.at[pl.ds(base + k * CW, CW)], ring.at[b],
                             semL.at[b])
        for b in range(RB):
            k = g * RB + b
            pltpu.make_async_copy(upd.at[pl.ds(base, CW)], ring.at[b],
                                  semL.at[b]).wait()
            pltpu.async_copy(ring.at[b], x.at[idx_v.at[k]], semA.at[b])
        return 0

    lax.fori_loop(0, NG, group, 0)
    for b in range(RB):
        pltpu.make_async_copy(ring.at[b], x.at[idx_v.at[0]], semA.at[b]).wait()


def _spec_a():
    return pl.BlockSpec((BLK, D), lambda i: (i, 0))


def _spec_b():
    return pl.BlockSpec((BLK, D), lambda i: ((i + HB) % NB, 0))


def _wspec(shape):
    nd = len(shape)
    return pl.BlockSpec(shape, lambda i: (0,) * nd)


def _tc_msgs_body(xa, xb, w1, b1, w2, b2, o):
    a = jnp.concatenate([xa[...], xb[...]], axis=1)
    h = jnp.maximum(jnp.dot(a, w1[...], preferred_element_type=jnp.float32)
                    + b1[...], 0.0)
    o[...] = jnp.dot(h, w2[...], preferred_element_type=jnp.float32) + b2[...]


def _tc_msgs(rows, W1, b1, W2, b2, swapped):
    first, second = (_spec_b(), _spec_a()) if swapped else (_spec_a(), _spec_b())
    return pl.pallas_call(
        _tc_msgs_body,
        grid=(NB,),
        in_specs=[first, second, _wspec((2 * D, H)), _wspec((1, H)),
                  _wspec((H, D)), _wspec((1, D))],
        out_specs=_spec_a(),
        out_shape=jax.ShapeDtypeStruct((TOT, D), jnp.float32),
    )(rows, rows, W1, b1, W2, b2)


def _tc_upd_body(xt, sv, cv, wu, bu, o):
    mean = sv[...] / jnp.maximum(cv[...][:, 0:1], 1.0)
    a = jnp.concatenate([xt[...], mean], axis=1)
    o[...] = jnp.maximum(
        jnp.dot(a, wu[...], preferred_element_type=jnp.float32) + bu[...], 0.0)


def _tc_upd(rows, sv, cv, Wu, bu, swapped):
    xt_spec = _spec_a() if swapped else _spec_b()
    return pl.pallas_call(
        _tc_upd_body,
        grid=(NB,),
        in_specs=[xt_spec, _spec_a(),
                  pl.BlockSpec((BLK, CVW), lambda i: (i, 0)),
                  _wspec((2 * D, D)), _wspec((1, D))],
        out_specs=_spec_a(),
        out_shape=jax.ShapeDtypeStruct((TOT, D), jnp.float32),
    )(rows, sv, cv, Wu, bu)


def _tc_pred_body(he, ae, w1, b1, w2, b2, w3, b3, o):
    a = jnp.concatenate([he[...], ae[...]], axis=1)
    p = jnp.maximum(jnp.dot(a, w1[...], preferred_element_type=jnp.float32)
                    + b1[...], 0.0)
    p = jnp.maximum(jnp.dot(p, w2[...], preferred_element_type=jnp.float32)
                    + b2[...], 0.0)
    z = jnp.dot(p, w3[...], preferred_element_type=jnp.float32) + b3[...]
    o[...] = 1.0 / (1.0 + jnp.exp(-z))


def _tc_pred(rows_ha, Wp1, bp1, Wp2, bp2, Wp3, bp3):
    nb = B // BLK
    he_spec = pl.BlockSpec((BLK, D), lambda i: (i, 0))
    ae_spec = pl.BlockSpec((BLK, D), lambda i: (i + nb, 0))
    return pl.pallas_call(
        _tc_pred_body,
        grid=(nb,),
        in_specs=[he_spec, ae_spec, _wspec((2 * D, H)), _wspec((1, H)),
                  _wspec((H, D)), _wspec((1, D)), _wspec((D, 1)),
                  _wspec((1, 1))],
        out_specs=pl.BlockSpec((BLK, 1), lambda i: (i, 0)),
        out_shape=jax.ShapeDtypeStruct((B, 1), jnp.float32),
    )(rows_ha, rows_ha, Wp1, bp1, Wp2, bp2, Wp3, bp3)


def kernel(emb, W1, b1, W2, b2, Wu, bu, Wp1, bp1, Wp2, bp2, Wp3, bp3,
           home_ids, away_ids, adjacency):
    adjacency = adjacency.astype(jnp.int32)
    home_ids = home_ids.astype(jnp.int32)
    away_ids = away_ids.astype(jnp.int32)
    src = adjacency[:, 0]
    dst = adjacency[:, 1]

    pad0 = (jnp.arange(EP - E, dtype=jnp.int32) * 523) % N_TEAMS
    padm = jnp.full((EP - E,), -1, jnp.int32)
    idxcat = jnp.concatenate([src, pad0, dst, pad0])
    tgt = jnp.concatenate([dst, padm, src, padm])

    in0 = (tgt >= 0) & (tgt < HALF)
    in1 = tgt >= HALF
    pos = jnp.arange(TOT, dtype=jnp.int32)
    slop = pos & (NSLOP - 1)
    tgt_all = jnp.stack([jnp.where(in0, tgt, DUMP + slop),
                         jnp.where(in1, tgt - HALF, DUMP + slop)])
    tgt_all = tgt_all.reshape(2, NSUB, SEG_CH, CW)
    pos_all = jnp.stack([jnp.where(in0, pos, OUT_DUMP + slop),
                         jnp.where(in1, pos, OUT_DUMP + slop)])
    pos_all = pos_all.reshape(2, NSUB, SEG_CH, CW)
    xtgt_all = jnp.stack([jnp.where(in0, tgt, XDUMP + slop),
                          jnp.where(in1, tgt, XDUMP + slop)])
    xtgt_all = xtgt_all.reshape(2, NSUB, SEG_CH, CW)
    idx3 = idxcat.reshape(NW, NCH, CW)
    ha3 = jnp.concatenate([home_ids, away_ids]).reshape(NW, 2 * B // NW // CW, CW)

    zrows = jnp.zeros((CW, D), jnp.float32)
    zrows16 = jnp.zeros((CW, CVW), jnp.float32)
    orows16 = jnp.ones((CW, CVW), jnp.float32)
    b1r = b1.reshape(1, H)
    b2r = b2.reshape(1, D)
    bur = bu.reshape(1, D)
    bp1r = bp1.reshape(1, H)
    bp2r = bp2.reshape(1, D)
    bp3r = bp3.reshape(1, 1)

    rows = _sc_gather_edges(emb, idx3)
    cv = _sc_cnt(zrows16, orows16, tgt_all, pos_all)

    swapped = False
    for _ in range(PASSES):
        msgs = _tc_msgs(rows, W1, b1r, W2, b2r, swapped)
        sv = _sc_segsum(msgs, zrows, tgt_all, pos_all)
        rows = _tc_upd(rows, sv, cv, Wu, bur, swapped)
        swapped = True

    x = _sc_buildx(emb, rows, xtgt_all)
    rows_ha = _sc_gather_pairs(x, ha3)
    return _tc_pred(rows_ha, Wp1, bp1r, Wp2, bp2r, Wp3, bp3r)


# speedup vs baseline: 4.6439x; 1.0801x over previous
import functools

import jax
import jax.numpy as jnp
from jax import lax
from jax.experimental import pallas as pl
from jax.experimental.pallas import tpu as pltpu
from jax.experimental.pallas import tpu_sc as plsc

N_TEAMS = 100000
D = 32
H = 64
E = 20000
B = 16384
PASSES = 3

EP = 20480
TOT = 2 * EP
NW = 32
NSUB = 16
HALF = 50000
NSLOP = 64
DUMP = HALF
SUMROWS = HALF + NSLOP
OUT_DUMP = TOT
XDUMP = N_TEAMS
XROWS = N_TEAMS + NSLOP
CW = 128
NCH = TOT // NW // CW
CHUNK = NCH * CW
SEG_CH = TOT // NSUB // CW
SEG_ROWS = SEG_CH * CW
RB = 5
NG = SEG_CH // RB

_MESH = plsc.VectorSubcoreMesh(core_axis_name="c", subcore_axis_name="s")

BLK = 2048
NB = TOT // BLK
HB = EP // BLK


def _make_sc_gather(nch):
    rows_per_tile = nch * CW

    @functools.partial(
        pl.kernel,
        out_type=jax.ShapeDtypeStruct((NW * rows_per_tile, D), jnp.float32),
        mesh=_MESH,
        compiler_params=pltpu.CompilerParams(use_tc_tiling_on_sc=False),
        scratch_types=[
            pltpu.VMEM((nch, CW), jnp.int32),
            pltpu.VMEM((rows_per_tile, D), jnp.float32),
            pltpu.SemaphoreType.DMA,
        ],
    )
    def k(table, idx3, out, idx_v, buf, sem):
        c = lax.axis_index("c")
        s = lax.axis_index("s")
        w = c * NSUB + s
        pltpu.sync_copy(idx3.at[w], idx_v)

        def fire(kk, _):
            pltpu.async_copy(table.at[idx_v.at[kk]],
                             buf.at[pl.ds(kk * CW, CW)], sem)
            return 0

        lax.fori_loop(0, nch, fire, 0)

        def drain(kk, _):
            pltpu.make_async_copy(table.at[idx_v.at[0]],
                                  buf.at[pl.ds(0, CW)], sem).wait()
            return 0

        lax.fori_loop(0, nch, drain, 0)
        pltpu.sync_copy(buf, out.at[pl.ds(w * rows_per_tile, rows_per_tile)])

    return k


_sc_gather_edges = _make_sc_gather(NCH)
_sc_gather_pairs = _make_sc_gather(2 * B // NW // CW)


@functools.partial(
    pl.kernel,
    out_type=jax.ShapeDtypeStruct((TOT + NSLOP, D), jnp.float32),
    mesh=_MESH,
    compiler_params=pltpu.CompilerParams(use_tc_tiling_on_sc=False),
    scratch_types=[
        pltpu.VMEM_SHARED((SUMROWS, D), jnp.float32),
        pltpu.VMEM((SEG_CH, CW), jnp.int32),
        pltpu.VMEM((SEG_CH, CW), jnp.int32),
        pltpu.VMEM((RB + 1, CW, D), jnp.float32),
        pltpu.SemaphoreType.DMA((RB,)),
        pltpu.SemaphoreType.DMA((RB,)),
    ],
)
def _sc_segsum(msgs, zrows, tgt_all, pos_all, out, sums, idx_v, pos_v, ring,
               semL, semA):
    c = lax.axis_index("c")
    s = lax.axis_index("s")
    pltpu.sync_copy(tgt_all.at[c, s], idx_v)
    pltpu.sync_copy(pos_all.at[c, s], pos_v)
    base = s * SEG_ROWS

    pltpu.sync_copy(zrows, ring.at[RB])
    for b in range(RB):
        pltpu.async_copy(msgs.at[pl.ds(base + b * CW, CW)], ring.at[b],
                         semL.at[b])

    def zfire(kk, _):
        pltpu.async_copy(ring.at[RB], sums.at[idx_v.at[kk]], semA.at[0])
        return 0

    lax.fori_loop(0, SEG_CH, zfire, 0)

    def zdrain(kk, _):
        pltpu.make_async_copy(ring.at[RB], sums.at[idx_v.at[0]],
                              semA.at[0]).wait()
        return 0

    lax.fori_loop(0, SEG_CH, zdrain, 0)
    plsc.subcore_barrier()

    def agroup(g, _):
        for b in range(RB):
            k = g * RB + b
            pltpu.make_async_copy(msgs.at[pl.ds(base, CW)], ring.at[b],
                                  semL.at[b]).wait()
            pltpu.async_copy(ring.at[b], sums.at[idx_v.at[k]], semA.at[b],
                             add=True)
        for b in range(RB):
            k = g * RB + b

            @pl.when(g < NG - 1)
            def _prefire_next():
                pltpu.make_async_copy(ring.at[b], sums.at[idx_v.at[0]],
                                      semA.at[b]).wait()
                pltpu.async_copy(msgs.at[pl.ds(base + (k + RB) * CW, CW)],
                                 ring.at[b], semL.at[b])

        return 0

    lax.fori_loop(0, NG, agroup, 0)
    for b in range(RB):
        pltpu.make_async_copy(ring.at[b], sums.at[idx_v.at[0]],
                              semA.at[b]).wait()
    plsc.subcore_barrier()

    def bgroup(g, _):
        for b in range(RB):
            k = g * RB + b

            @pl.when(g > 0)
            def _wait_out():
                pltpu.make_async_copy(ring.at[b], out.at[pos_v.at[0]],
                                      semA.at[b]).wait()

            pltpu.async_copy(sums.at[idx_v.at[k]], ring.at[b], semL.at[b])
        for b in range(RB):
            k = g * RB + b
            pltpu.make_async_copy(sums.at[idx_v.at[0]], ring.at[b],
                                  semL.at[b]).wait()
            pltpu.async_copy(ring.at[b], out.at[pos_v.at[k]], semA.at[b])
        return 0

    lax.fori_loop(0, NG, bgroup, 0)
    for b in range(RB):
        pltpu.make_async_copy(ring.at[b], out.at[pos_v.at[0]],
                              semA.at[b]).wait()


CVW = 16


@functools.partial(
    pl.kernel,
    out_type=jax.ShapeDtypeStruct((TOT + NSLOP, CVW), jnp.float32),
    mesh=_MESH,
    compiler_params=pltpu.CompilerParams(use_tc_tiling_on_sc=False),
    scratch_types=[
        pltpu.VMEM_SHARED((SUMROWS, CVW), jnp.float32),
        pltpu.VMEM((SEG_CH, CW), jnp.int32),
        pltpu.VMEM((SEG_CH, CW), jnp.int32),
        pltpu.VMEM((RB, CW, CVW), jnp.float32),
        pltpu.SemaphoreType.DMA((RB,)),
        pltpu.SemaphoreType.DMA((RB,)),
    ],
)
def _sc_cnt(zrows, orows, tgt_all, pos_all, out, cnts, idx_v, pos_v, ring,
            semL, semA):
    c = lax.axis_index("c")
    s = lax.axis_index("s")
    pltpu.sync_copy(tgt_all.at[c, s], idx_v)
    pltpu.sync_copy(pos_all.at[c, s], pos_v)
    pltpu.sync_copy(zrows, ring.at[0])
    pltpu.sync_copy(orows, ring.at[1])

    def zfire(kk, _):
        pltpu.async_copy(ring.at[0], cnts.at[idx_v.at[kk]], semA.at[0])
        return 0

    lax.fori_loop(0, SEG_CH, zfire, 0)

    def zdrain(kk, _):
        pltpu.make_async_copy(ring.at[0], cnts.at[idx_v.at[0]],
                              semA.at[0]).wait()
        return 0

    lax.fori_loop(0, SEG_CH, zdrain, 0)
    plsc.subcore_barrier()

    def afire(kk, _):
        pltpu.async_copy(ring.at[1], cnts.at[idx_v.at[kk]], semA.at[1],
                         add=True)
        return 0

    lax.fori_loop(0, SEG_CH, afire, 0)

    def adrain(kk, _):
        pltpu.make_async_copy(ring.at[1], cnts.at[idx_v.at[0]],
                              semA.at[1]).wait()
        return 0

    lax.fori_loop(0, SEG_CH, adrain, 0)
    plsc.subcore_barrier()

    def bgroup(g, _):
        for b in range(RB):
            k = g * RB + b

            @pl.when(g > 0)
            def _wait_out():
                pltpu.make_async_copy(ring.at[b], out.at[pos_v.at[0]],
                                      semA.at[b]).wait()

            pltpu.async_copy(cnts.at[idx_v.at[k]], ring.at[b], semL.at[b])
        for b in range(RB):
            k = g * RB + b
            pltpu.make_async_copy(cnts.at[idx_v.at[0]], ring.at[b],
                                  semL.at[b]).wait()
            pltpu.async_copy(ring.at[b], out.at[pos_v.at[k]], semA.at[b])
        return 0

    lax.fori_loop(0, NG, bgroup, 0)
    for b in range(RB):
        pltpu.make_async_copy(ring.at[b], out.at[pos_v.at[0]],
                              semA.at[b]).wait()


@functools.partial(
    pl.kernel,
    out_type=jax.ShapeDtypeStruct((XROWS, D), jnp.float32),
    mesh=_MESH,
    compiler_params=pltpu.CompilerParams(use_tc_tiling_on_sc=False),
    scratch_types=[
        pltpu.VMEM((HALF // NSUB, D), jnp.float32),
        pltpu.VMEM((SEG_CH, CW), jnp.int32),
        pltpu.VMEM((RB, CW, D), jnp.float32),
        pltpu.SemaphoreType.DMA((RB,)),
        pltpu.SemaphoreType.DMA((RB,)),
    ],
)
def _sc_buildx(emb, upd, xtgt_all, x, cbuf, idx_v, ring, semL, semA):
    c = lax.axis_index("c")
    s = lax.axis_index("s")
    tbase = c * HALF + s * (HALF // NSUB)
    pltpu.sync_copy(emb.at[pl.ds(tbase, HALF // NSUB)], cbuf)
    pltpu.sync_copy(cbuf, x.at[pl.ds(tbase, HALF // NSUB)])
    plsc.subcore_barrier()

    pltpu.sync_copy(xtgt_all.at[c, s], idx_v)
    base = s * SEG_ROWS

    def group(g, _):
        for b in range(RB):
            k = g * RB + b

            @pl.when(g > 0)
            def _wait_scat():
                pltpu.make_async_copy(ring.at[b], x.at[idx_v.at[0]],
                                      semA.at[b]).wait()

            pltpu.async_copy(upd.at[pl.ds(base + k * CW, CW)], ring.at[b],
                             semL.at[b])
        for b in range(RB):
            k = g * RB + b
            pltpu.make_async_copy(upd.at[pl.ds(base, CW)], ring.at[b],
                                  semL.at[b]).wait()
            pltpu.async_copy(ring.at[b], x.at[idx_v.at[k]], semA.at[b])
        return 0

    lax.fori_loop(0, NG, group, 0)
    for b in range(RB):
        pltpu.make_async_copy(ring.at[b], x.at[idx_v.at[0]], semA.at[b]).wait()


PG = 4
PR = TOT // PG
PBLK = BLK // PG
NPB = PR // PBLK
HPB = (EP // PG) // PBLK


def _alt_pair(W, n_out):
    a = W.shape[0] // 2
    top, bot = W[:a], W[a:]
    Z = jnp.zeros((PG * 2 * a, PG * n_out), jnp.float32)
    for g in range(PG):
        Z = Z.at[g * a:(g + 1) * a, g * n_out:(g + 1) * n_out].set(top)
        Z = Z.at[PG * a + g * a:PG * a + (g + 1) * a,
                 g * n_out:(g + 1) * n_out].set(bot)
    return Z


def _bdiag(W):
    k, n = W.shape
    Z = jnp.zeros((PG * k, PG * n), jnp.float32)
    for g in range(PG):
        Z = Z.at[g * k:(g + 1) * k, g * n:(g + 1) * n].set(W)
    return Z


def _spec_ap():
    return pl.BlockSpec((PBLK, PG * D), lambda i: (i, 0))


def _spec_bp():
    return pl.BlockSpec((PBLK, PG * D), lambda i: ((i + HPB) % NPB, 0))


def _wspec(shape):
    nd = len(shape)
    return pl.BlockSpec(shape, lambda i: (0,) * nd)


def _tc_msgs_body(xa, xb, r, w1, b1, w2, b2, o):
    a = jnp.concatenate([xa[...], xb[...]], axis=1)
    h = jnp.maximum(jnp.dot(a, w1[...], preferred_element_type=jnp.float32)
                    + b1[...], 0.0)
    m = jnp.dot(h, w2[...], preferred_element_type=jnp.float32) + b2[...]
    o[...] = m * r[...]


def _tc_msgs(rows_p, r_p, W1a, b1t, W2d, b2t, swapped):
    first, second = ((_spec_bp(), _spec_ap()) if swapped
                     else (_spec_ap(), _spec_bp()))
    return pl.pallas_call(
        _tc_msgs_body,
        grid=(NPB,),
        in_specs=[first, second, _spec_ap(), _wspec((PG * 2 * D, PG * H)),
                  _wspec((1, PG * H)), _wspec((PG * H, PG * D)),
                  _wspec((1, PG * D))],
        out_specs=pl.BlockSpec((PBLK, PG * D), lambda i: (i, 0)),
        out_shape=jax.ShapeDtypeStruct((PR, PG * D), jnp.float32),
    )(rows_p, rows_p, r_p, W1a, b1t, W2d, b2t)


def _tc_upd_body(xt, sv, wu, bu, o):
    a = jnp.concatenate([xt[...], sv[...]], axis=1)
    o[...] = jnp.maximum(
        jnp.dot(a, wu[...], preferred_element_type=jnp.float32) + bu[...], 0.0)


def _tc_upd(rows_p, sv_p, Wua, but, swapped):
    xt_spec = _spec_ap() if swapped else _spec_bp()
    return pl.pallas_call(
        _tc_upd_body,
        grid=(NPB,),
        in_specs=[xt_spec, pl.BlockSpec((PBLK, PG * D), lambda i: (i, 0)),
                  _wspec((PG * 2 * D, PG * D)), _wspec((1, PG * D))],
        out_specs=pl.BlockSpec((PBLK, PG * D), lambda i: (i, 0)),
        out_shape=jax.ShapeDtypeStruct((PR, PG * D), jnp.float32),
    )(rows_p, sv_p, Wua, but)


def _tc_pred_body(he, ae, w1, b1, w2, b2, w3, b3, o):
    a = jnp.concatenate([he[...], ae[...]], axis=1)
    p = jnp.maximum(jnp.dot(a, w1[...], preferred_element_type=jnp.float32)
                    + b1[...], 0.0)
    p = jnp.maximum(jnp.dot(p, w2[...], preferred_element_type=jnp.float32)
                    + b2[...], 0.0)
    z = jnp.dot(p, w3[...], preferred_element_type=jnp.float32) + b3[...]
    o[...] = 1.0 / (1.0 + jnp.exp(-z))


def _tc_pred(ha_p, Wp1a, bp1t, Wp2d, bp2t, Wp3d, bp3t):
    npb = (B // PG) // PBLK
    he_spec = pl.BlockSpec((PBLK, PG * D), lambda i: (i, 0))
    ae_spec = pl.BlockSpec((PBLK, PG * D), lambda i: (i + npb, 0))
    return pl.pallas_call(
        _tc_pred_body,
        grid=(npb,),
        in_specs=[he_spec, ae_spec, _wspec((PG * 2 * D, PG * H)),
                  _wspec((1, PG * H)), _wspec((PG * H, PG * D)),
                  _wspec((1, PG * D)), _wspec((PG * D, PG)),
                  _wspec((1, PG))],
        out_specs=pl.BlockSpec((PBLK, PG), lambda i: (i, 0)),
        out_shape=jax.ShapeDtypeStruct((B // PG, PG), jnp.float32),
    )(ha_p, ha_p, Wp1a, bp1t, Wp2d, bp2t, Wp3d, bp3t)


def kernel(emb, W1, b1, W2, b2, Wu, bu, Wp1, bp1, Wp2, bp2, Wp3, bp3,
           home_ids, away_ids, adjacency):
    adjacency = adjacency.astype(jnp.int32)
    home_ids = home_ids.astype(jnp.int32)
    away_ids = away_ids.astype(jnp.int32)
    src = adjacency[:, 0]
    dst = adjacency[:, 1]

    pad0 = (jnp.arange(EP - E, dtype=jnp.int32) * 523) % N_TEAMS
    padm = jnp.full((EP - E,), -1, jnp.int32)
    idxcat = jnp.concatenate([src, pad0, dst, pad0])
    tgt = jnp.concatenate([dst, padm, src, padm])

    in0 = (tgt >= 0) & (tgt < HALF)
    in1 = tgt >= HALF
    pos = jnp.arange(TOT, dtype=jnp.int32)
    slop = pos & (NSLOP - 1)
    tgt_all = jnp.stack([jnp.where(in0, tgt, DUMP + slop),
                         jnp.where(in1, tgt - HALF, DUMP + slop)])
    tgt_all = tgt_all.reshape(2, NSUB, SEG_CH, CW)
    pos_all = jnp.stack([jnp.where(in0, pos, OUT_DUMP + slop),
                         jnp.where(in1, pos, OUT_DUMP + slop)])
    pos_all = pos_all.reshape(2, NSUB, SEG_CH, CW)
    xtgt_all = jnp.stack([jnp.where(in0, tgt, XDUMP + slop),
                          jnp.where(in1, tgt, XDUMP + slop)])
    xtgt_all = xtgt_all.reshape(2, NSUB, SEG_CH, CW)
    idx3 = idxcat.reshape(NW, NCH, CW)
    ha3 = jnp.concatenate([home_ids, away_ids]).reshape(NW, 2 * B // NW // CW, CW)

    zrows = jnp.zeros((CW, D), jnp.float32)
    zrows16 = jnp.zeros((CW, CVW), jnp.float32)
    orows16 = jnp.ones((CW, CVW), jnp.float32)
    W1a = _alt_pair(W1, H)
    b1t = jnp.tile(b1, PG).reshape(1, PG * H)
    W2d = _bdiag(W2)
    b2t = jnp.tile(b2, PG).reshape(1, PG * D)
    Wua = _alt_pair(Wu, D)
    but = jnp.tile(bu, PG).reshape(1, PG * D)
    Wp1a = _alt_pair(Wp1, H)
    bp1t = jnp.tile(bp1, PG).reshape(1, PG * H)
    Wp2d = _bdiag(Wp2)
    bp2t = jnp.tile(bp2, PG).reshape(1, PG * D)
    Wp3d = _bdiag(Wp3)
    bp3t = jnp.tile(bp3, PG).reshape(1, PG)

    rows = _sc_gather_edges(emb, idx3)
    cv = _sc_cnt(zrows16, orows16, tgt_all, pos_all)
    r_p = jnp.broadcast_to(1.0 / jnp.maximum(cv[:TOT, 0:1], 1.0),
                           (TOT, D)).reshape(PR, PG * D)

    rows_p = rows.reshape(PR, PG * D)
    swapped = False
    for _ in range(PASSES):
        msgs_p = _tc_msgs(rows_p, r_p, W1a, b1t, W2d, b2t, swapped)
        sv = _sc_segsum(msgs_p.reshape(TOT, D), zrows, tgt_all, pos_all)
        sv_p = sv[:TOT].reshape(PR, PG * D)
        rows_p = _tc_upd(rows_p, sv_p, Wua, but, swapped)
        swapped = True

    x = _sc_buildx(emb, rows_p.reshape(TOT, D), xtgt_all)
    rows_ha = _sc_gather_pairs(x, ha3)
    ha_p = rows_ha.reshape(2 * B // PG, PG * D)
    return _tc_pred(ha_p, Wp1a, bp1t, Wp2d, bp2t, Wp3d, bp3t).reshape(B, 1)


# speedup vs baseline: 6.2845x; 1.3533x over previous
import functools

import jax
import jax.numpy as jnp
from jax import lax
from jax.experimental import pallas as pl
from jax.experimental.pallas import tpu as pltpu
from jax.experimental.pallas import tpu_sc as plsc

N_TEAMS = 100000
D = 32
H = 64
E = 20000
B = 16384
PASSES = 3

EP = 20480
TOT = 2 * EP
NW = 32
NSUB = 16
HALF = 50000
NSLOP = 64
DUMP = HALF
SUMROWS = HALF + NSLOP
OUT_DUMP = TOT
XDUMP = N_TEAMS
XROWS = N_TEAMS + NSLOP
CW = 128
NCH = TOT // NW // CW
CHUNK = NCH * CW
SEG_CH = TOT // NSUB // CW
SEG_ROWS = SEG_CH * CW
RB = 5
NG = SEG_CH // RB

_MESH = plsc.VectorSubcoreMesh(core_axis_name="c", subcore_axis_name="s")

BLK = 2048
NB = TOT // BLK
HB = EP // BLK


def _make_sc_gather(nch):
    rows_per_tile = nch * CW

    @functools.partial(
        pl.kernel,
        out_type=jax.ShapeDtypeStruct((NW * rows_per_tile, D), jnp.float32),
        mesh=_MESH,
        compiler_params=pltpu.CompilerParams(use_tc_tiling_on_sc=False),
        scratch_types=[
            pltpu.VMEM((nch, CW), jnp.int32),
            pltpu.VMEM((rows_per_tile, D), jnp.float32),
            pltpu.SemaphoreType.DMA,
        ],
    )
    def k(table, idx3, out, idx_v, buf, sem):
        c = lax.axis_index("c")
        s = lax.axis_index("s")
        w = c * NSUB + s
        pltpu.sync_copy(idx3.at[w], idx_v)

        def fire(kk, _):
            pltpu.async_copy(table.at[idx_v.at[kk]],
                             buf.at[pl.ds(kk * CW, CW)], sem)
            return 0

        lax.fori_loop(0, nch, fire, 0)

        def drain(kk, _):
            pltpu.make_async_copy(table.at[idx_v.at[0]],
                                  buf.at[pl.ds(0, CW)], sem).wait()
            return 0

        lax.fori_loop(0, nch, drain, 0)
        pltpu.sync_copy(buf, out.at[pl.ds(w * rows_per_tile, rows_per_tile)])

    return k


_sc_gather_edges = _make_sc_gather(NCH)
_sc_gather_pairs = _make_sc_gather(2 * B // NW // CW)


@functools.partial(
    pl.kernel,
    out_type=jax.ShapeDtypeStruct((TOT + NSLOP, D), jnp.float32),
    mesh=_MESH,
    compiler_params=pltpu.CompilerParams(use_tc_tiling_on_sc=False),
    scratch_types=[
        pltpu.VMEM_SHARED((SUMROWS, D), jnp.float32),
        pltpu.VMEM((SEG_CH, CW), jnp.int32),
        pltpu.VMEM((SEG_CH, CW), jnp.int32),
        pltpu.VMEM((RB + 1, CW, D), jnp.float32),
        pltpu.SemaphoreType.DMA((RB,)),
        pltpu.SemaphoreType.DMA((RB,)),
    ],
)
def _sc_segsum(msgs, zrows, tgt_all, pos_all, out, sums, idx_v, pos_v, ring,
               semL, semA):
    c = lax.axis_index("c")
    s = lax.axis_index("s")
    pltpu.sync_copy(tgt_all.at[c, s], idx_v)
    pltpu.sync_copy(pos_all.at[c, s], pos_v)
    base = s * SEG_ROWS

    pltpu.sync_copy(zrows, ring.at[RB])
    for b in range(RB):
        pltpu.async_copy(msgs.at[pl.ds(base + b * CW, CW)], ring.at[b],
                         semL.at[b])

    def zfire(kk, _):
        pltpu.async_copy(ring.at[RB], sums.at[idx_v.at[kk]], semA.at[0])
        return 0

    lax.fori_loop(0, SEG_CH, zfire, 0)

    def zdrain(kk, _):
        pltpu.make_async_copy(ring.at[RB], sums.at[idx_v.at[0]],
                              semA.at[0]).wait()
        return 0

    lax.fori_loop(0, SEG_CH, zdrain, 0)
    plsc.subcore_barrier()

    def agroup(g, _):
        for b in range(RB):
            k = g * RB + b
            pltpu.make_async_copy(msgs.at[pl.ds(base, CW)], ring.at[b],
                                  semL.at[b]).wait()
            pltpu.async_copy(ring.at[b], sums.at[idx_v.at[k]], semA.at[b],
                             add=True)
        for b in range(RB):
            k = g * RB + b

            @pl.when(g < NG - 1)
            def _prefire_next():
                pltpu.make_async_copy(ring.at[b], sums.at[idx_v.at[0]],
                                      semA.at[b]).wait()
                pltpu.async_copy(msgs.at[pl.ds(base + (k + RB) * CW, CW)],
                                 ring.at[b], semL.at[b])

        return 0

    lax.fori_loop(0, NG, agroup, 0)
    for b in range(RB):
        pltpu.make_async_copy(ring.at[b], sums.at[idx_v.at[0]],
                              semA.at[b]).wait()
    plsc.subcore_barrier()

    def bgroup(g, _):
        for b in range(RB):
            k = g * RB + b

            @pl.when(g > 0)
            def _wait_out():
                pltpu.make_async_copy(ring.at[b], out.at[pos_v.at[0]],
                                      semA.at[b]).wait()

            pltpu.async_copy(sums.at[idx_v.at[k]], ring.at[b], semL.at[b])
        for b in range(RB):
            k = g * RB + b
            pltpu.make_async_copy(sums.at[idx_v.at[0]], ring.at[b],
                                  semL.at[b]).wait()
            pltpu.async_copy(ring.at[b], out.at[pos_v.at[k]], semA.at[b])
        return 0

    lax.fori_loop(0, NG, bgroup, 0)
    for b in range(RB):
        pltpu.make_async_copy(ring.at[b], out.at[pos_v.at[0]],
                              semA.at[b]).wait()


CVW = 16


@functools.partial(
    pl.kernel,
    out_type=jax.ShapeDtypeStruct((TOT + NSLOP, CVW), jnp.float32),
    mesh=_MESH,
    compiler_params=pltpu.CompilerParams(use_tc_tiling_on_sc=False),
    scratch_types=[
        pltpu.VMEM_SHARED((SUMROWS, CVW), jnp.float32),
        pltpu.VMEM((SEG_CH, CW), jnp.int32),
        pltpu.VMEM((SEG_CH, CW), jnp.int32),
        pltpu.VMEM((RB, CW, CVW), jnp.float32),
        pltpu.SemaphoreType.DMA((RB,)),
        pltpu.SemaphoreType.DMA((RB,)),
    ],
)
def _sc_cnt(zrows, orows, tgt_all, pos_all, out, cnts, idx_v, pos_v, ring,
            semL, semA):
    c = lax.axis_index("c")
    s = lax.axis_index("s")
    pltpu.sync_copy(tgt_all.at[c, s], idx_v)
    pltpu.sync_copy(pos_all.at[c, s], pos_v)
    pltpu.sync_copy(zrows, ring.at[0])
    pltpu.sync_copy(orows, ring.at[1])

    def zfire(kk, _):
        pltpu.async_copy(ring.at[0], cnts.at[idx_v.at[kk]], semA.at[0])
        return 0

    lax.fori_loop(0, SEG_CH, zfire, 0)

    def zdrain(kk, _):
        pltpu.make_async_copy(ring.at[0], cnts.at[idx_v.at[0]],
                              semA.at[0]).wait()
        return 0

    lax.fori_loop(0, SEG_CH, zdrain, 0)
    plsc.subcore_barrier()

    def afire(kk, _):
        pltpu.async_copy(ring.at[1], cnts.at[idx_v.at[kk]], semA.at[1],
                         add=True)
        return 0

    lax.fori_loop(0, SEG_CH, afire, 0)

    def adrain(kk, _):
        pltpu.make_async_copy(ring.at[1], cnts.at[idx_v.at[0]],
                              semA.at[1]).wait()
        return 0

    lax.fori_loop(0, SEG_CH, adrain, 0)
    plsc.subcore_barrier()

    def bgroup(g, _):
        for b in range(RB):
            k = g * RB + b

            @pl.when(g > 0)
            def _wait_out():
                pltpu.make_async_copy(ring.at[b], out.at[pos_v.at[0]],
                                      semA.at[b]).wait()

            pltpu.async_copy(cnts.at[idx_v.at[k]], ring.at[b], semL.at[b])
        for b in range(RB):
            k = g * RB + b
            pltpu.make_async_copy(cnts.at[idx_v.at[0]], ring.at[b],
                                  semL.at[b]).wait()
            pltpu.async_copy(ring.at[b], out.at[pos_v.at[k]], semA.at[b])
        return 0

    lax.fori_loop(0, NG, bgroup, 0)
    for b in range(RB):
        pltpu.make_async_copy(ring.at[b], out.at[pos_v.at[0]],
                              semA.at[b]).wait()


@functools.partial(
    pl.kernel,
    out_type=jax.ShapeDtypeStruct((XROWS, D), jnp.float32),
    mesh=_MESH,
    compiler_params=pltpu.CompilerParams(use_tc_tiling_on_sc=False),
    scratch_types=[
        pltpu.VMEM((HALF // NSUB, D), jnp.float32),
        pltpu.VMEM((SEG_CH, CW), jnp.int32),
        pltpu.VMEM((RB, CW, D), jnp.float32),
        pltpu.SemaphoreType.DMA((RB,)),
        pltpu.SemaphoreType.DMA((RB,)),
    ],
)
def _sc_buildx(emb, upd, xtgt_all, x, cbuf, idx_v, ring, semL, semA):
    c = lax.axis_index("c")
    s = lax.axis_index("s")
    tbase = c * HALF + s * (HALF // NSUB)
    pltpu.sync_copy(emb.at[pl.ds(tbase, HALF // NSUB)], cbuf)
    pltpu.sync_copy(cbuf, x.at[pl.ds(tbase, HALF // NSUB)])
    plsc.subcore_barrier()

    pltpu.sync_copy(xtgt_all.at[c, s], idx_v)
    base = s * SEG_ROWS

    def group(g, _):
        for b in range(RB):
            k = g * RB + b

            @pl.when(g > 0)
            def _wait_scat():
                pltpu.make_async_copy(ring.at[b], x.at[idx_v.at[0]],
                                      semA.at[b]).wait()

            pltpu.async_copy(upd.at[pl.ds(base + k * CW, CW)], ring.at[b],
                             semL.at[b])
        for b in range(RB):
            k = g * RB + b
            pltpu.make_async_copy(upd.at[pl.ds(base, CW)], ring.at[b],
                                  semL.at[b]).wait()
            pltpu.async_copy(ring.at[b], x.at[idx_v.at[k]], semA.at[b])
        return 0

    lax.fori_loop(0, NG, group, 0)
    for b in range(RB):
        pltpu.make_async_copy(ring.at[b], x.at[idx_v.at[0]], semA.at[b]).wait()


PG = 4
PR = TOT // PG
PBLK = BLK // PG
NPB = PR // PBLK
HPB = (EP // PG) // PBLK


def _alt_pair(W, n_out):
    a = W.shape[0] // 2
    top, bot = W[:a], W[a:]
    Z = jnp.zeros((PG * 2 * a, PG * n_out), jnp.float32)
    for g in range(PG):
        Z = Z.at[g * a:(g + 1) * a, g * n_out:(g + 1) * n_out].set(top)
        Z = Z.at[PG * a + g * a:PG * a + (g + 1) * a,
                 g * n_out:(g + 1) * n_out].set(bot)
    return Z


def _bdiag(W):
    k, n = W.shape
    Z = jnp.zeros((PG * k, PG * n), jnp.float32)
    for g in range(PG):
        Z = Z.at[g * k:(g + 1) * k, g * n:(g + 1) * n].set(W)
    return Z


def _spec_ap():
    return pl.BlockSpec((PBLK, PG * D), lambda i: (i, 0))


def _spec_bp():
    return pl.BlockSpec((PBLK, PG * D), lambda i: ((i + HPB) % NPB, 0))


def _wspec(shape):
    nd = len(shape)
    return pl.BlockSpec(shape, lambda i: (0,) * nd)


def _tc_msgs_body(xa, xb, w1, b1, w2, b2, o):
    a = jnp.concatenate([xa[...], xb[...]], axis=1)
    h = jnp.maximum(jnp.dot(a, w1[...], preferred_element_type=jnp.float32)
                    + b1[...], 0.0)
    o[...] = jnp.dot(h, w2[...], preferred_element_type=jnp.float32) + b2[...]


def _tc_msgs(rows_p, W1a, b1t, W2d, b2t, swapped):
    first, second = ((_spec_bp(), _spec_ap()) if swapped
                     else (_spec_ap(), _spec_bp()))
    return pl.pallas_call(
        _tc_msgs_body,
        grid=(NPB,),
        in_specs=[first, second, _wspec((PG * 2 * D, PG * H)),
                  _wspec((1, PG * H)), _wspec((PG * H, PG * D)),
                  _wspec((1, PG * D))],
        out_specs=pl.BlockSpec((PBLK, PG * D), lambda i: (i, 0)),
        out_shape=jax.ShapeDtypeStruct((PR, PG * D), jnp.float32),
    )(rows_p, rows_p, W1a, b1t, W2d, b2t)


def _tc_upd_body(xt, sv, r, wu, bu, o):
    a = jnp.concatenate([xt[...], sv[...] * r[...]], axis=1)
    o[...] = jnp.maximum(
        jnp.dot(a, wu[...], preferred_element_type=jnp.float32) + bu[...], 0.0)


def _tc_upd(rows_p, sv_p, r_p, Wua, but, swapped):
    xt_spec = _spec_ap() if swapped else _spec_bp()
    return pl.pallas_call(
        _tc_upd_body,
        grid=(NPB,),
        in_specs=[xt_spec, pl.BlockSpec((PBLK, PG * D), lambda i: (i, 0)),
                  _spec_ap(), _wspec((PG * 2 * D, PG * D)),
                  _wspec((1, PG * D))],
        out_specs=pl.BlockSpec((PBLK, PG * D), lambda i: (i, 0)),
        out_shape=jax.ShapeDtypeStruct((PR, PG * D), jnp.float32),
    )(rows_p, sv_p, r_p, Wua, but)


def _tc_pred_body(he, ae, w1, b1, w2, b2, w3, b3, o):
    a = jnp.concatenate([he[...], ae[...]], axis=1)
    p = jnp.maximum(jnp.dot(a, w1[...], preferred_element_type=jnp.float32)
                    + b1[...], 0.0)
    p = jnp.maximum(jnp.dot(p, w2[...], preferred_element_type=jnp.float32)
                    + b2[...], 0.0)
    z = jnp.dot(p, w3[...], preferred_element_type=jnp.float32) + b3[...]
    o[...] = 1.0 / (1.0 + jnp.exp(-z))


def _tc_pred(ha_p, Wp1a, bp1t, Wp2d, bp2t, Wp3d, bp3t):
    npb = (B // PG) // PBLK
    he_spec = pl.BlockSpec((PBLK, PG * D), lambda i: (i, 0))
    ae_spec = pl.BlockSpec((PBLK, PG * D), lambda i: (i + npb, 0))
    return pl.pallas_call(
        _tc_pred_body,
        grid=(npb,),
        in_specs=[he_spec, ae_spec, _wspec((PG * 2 * D, PG * H)),
                  _wspec((1, PG * H)), _wspec((PG * H, PG * D)),
                  _wspec((1, PG * D)), _wspec((PG * D, PG)),
                  _wspec((1, PG))],
        out_specs=pl.BlockSpec((PBLK, PG), lambda i: (i, 0)),
        out_shape=jax.ShapeDtypeStruct((B // PG, PG), jnp.float32),
    )(ha_p, ha_p, Wp1a, bp1t, Wp2d, bp2t, Wp3d, bp3t)


def kernel(emb, W1, b1, W2, b2, Wu, bu, Wp1, bp1, Wp2, bp2, Wp3, bp3,
           home_ids, away_ids, adjacency):
    adjacency = adjacency.astype(jnp.int32)
    home_ids = home_ids.astype(jnp.int32)
    away_ids = away_ids.astype(jnp.int32)
    src = adjacency[:, 0]
    dst = adjacency[:, 1]

    pad0 = (jnp.arange(EP - E, dtype=jnp.int32) * 523) % N_TEAMS
    padm = jnp.full((EP - E,), -1, jnp.int32)
    idxcat = jnp.concatenate([src, pad0, dst, pad0])
    tgt = jnp.concatenate([dst, padm, src, padm])

    in0 = (tgt >= 0) & (tgt < HALF)
    in1 = tgt >= HALF
    pos = jnp.arange(TOT, dtype=jnp.int32)
    slop = pos & (NSLOP - 1)
    tgt_all = jnp.stack([jnp.where(in0, tgt, DUMP + slop),
                         jnp.where(in1, tgt - HALF, DUMP + slop)])
    tgt_all = tgt_all.reshape(2, NSUB, SEG_CH, CW)
    pos_all = jnp.stack([jnp.where(in0, pos, OUT_DUMP + slop),
                         jnp.where(in1, pos, OUT_DUMP + slop)])
    pos_all = pos_all.reshape(2, NSUB, SEG_CH, CW)
    xtgt_all = jnp.stack([jnp.where(in0, tgt, XDUMP + slop),
                          jnp.where(in1, tgt, XDUMP + slop)])
    xtgt_all = xtgt_all.reshape(2, NSUB, SEG_CH, CW)
    idx3 = idxcat.reshape(NW, NCH, CW)
    ha3 = jnp.concatenate([home_ids, away_ids]).reshape(NW, 2 * B // NW // CW, CW)

    zrows = jnp.zeros((CW, D), jnp.float32)
    zrows16 = jnp.zeros((CW, CVW), jnp.float32)
    orows16 = jnp.ones((CW, CVW), jnp.float32)
    W1a = _alt_pair(W1, H)
    b1t = jnp.tile(b1, PG).reshape(1, PG * H)
    W2d = _bdiag(W2)
    b2t = jnp.tile(b2, PG).reshape(1, PG * D)
    Wua = _alt_pair(Wu, D)
    but = jnp.tile(bu, PG).reshape(1, PG * D)
    Wp1a = _alt_pair(Wp1, H)
    bp1t = jnp.tile(bp1, PG).reshape(1, PG * H)
    Wp2d = _bdiag(Wp2)
    bp2t = jnp.tile(bp2, PG).reshape(1, PG * D)
    Wp3d = _bdiag(Wp3)
    bp3t = jnp.tile(bp3, PG).reshape(1, PG)

    rows = _sc_gather_edges(emb, idx3)
    cv = _sc_cnt(zrows16, orows16, tgt_all, pos_all)
    r_p = jnp.broadcast_to(1.0 / jnp.maximum(cv[:TOT, 0:1], 1.0),
                           (TOT, D)).reshape(PR, PG * D)

    rows_p = rows.reshape(PR, PG * D)
    swapped = False
    for _ in range(PASSES):
        msgs_p = _tc_msgs(rows_p, W1a, b1t, W2d, b2t, swapped)
        sv = _sc_segsum(msgs_p.reshape(TOT, D), zrows, tgt_all, pos_all)
        sv_p = sv.reshape((TOT + NSLOP) // PG, PG * D)
        rows_p = _tc_upd(rows_p, sv_p, r_p, Wua, but, swapped)
        swapped = True

    x = _sc_buildx(emb, rows_p.reshape(TOT, D), xtgt_all)
    rows_ha = _sc_gather_pairs(x, ha3)
    ha_p = rows_ha.reshape(2 * B // PG, PG * D)
    return _tc_pred(ha_p, Wp1a, bp1t, Wp2d, bp2t, Wp3d, bp3t).reshape(B, 1)


# speedup vs baseline: 7.0922x; 1.1285x over previous
import functools

import jax
import jax.numpy as jnp
from jax import lax
from jax.experimental import pallas as pl
from jax.experimental.pallas import tpu as pltpu
from jax.experimental.pallas import tpu_sc as plsc

N_TEAMS = 100000
D = 32
H = 64
E = 20000
B = 16384
PASSES = 3

EP = 20480
TOT = 2 * EP
NW = 32
NSUB = 16
HALF = 50000
NSLOP = 128
DUMP = HALF
SUMROWS = HALF + NSLOP
OUT_DUMP = TOT
XDUMP = N_TEAMS
XROWS = N_TEAMS + NSLOP
CW = 128
NCH = TOT // NW // CW
CHUNK = NCH * CW
SEG_CH = TOT // NSUB // CW
SEG_ROWS = SEG_CH * CW
RB = 5
NG = SEG_CH // RB

_MESH = plsc.VectorSubcoreMesh(core_axis_name="c", subcore_axis_name="s")

BLK = 2048
NB = TOT // BLK
HB = EP // BLK


def _make_sc_gather(nch):
    rows_per_tile = nch * CW

    @functools.partial(
        pl.kernel,
        out_type=jax.ShapeDtypeStruct((NW * rows_per_tile, D), jnp.float32),
        mesh=_MESH,
        compiler_params=pltpu.CompilerParams(use_tc_tiling_on_sc=False),
        scratch_types=[
            pltpu.VMEM((nch, CW), jnp.int32),
            pltpu.VMEM((rows_per_tile, D), jnp.float32),
            pltpu.SemaphoreType.DMA,
        ],
    )
    def k(table, idx3, out, idx_v, buf, sem):
        c = lax.axis_index("c")
        s = lax.axis_index("s")
        w = c * NSUB + s
        pltpu.sync_copy(idx3.at[w], idx_v)

        def fire(kk, _):
            pltpu.async_copy(table.at[idx_v.at[kk]],
                             buf.at[pl.ds(kk * CW, CW)], sem)
            return 0

        lax.fori_loop(0, nch, fire, 0)

        def drain(kk, _):
            pltpu.make_async_copy(table.at[idx_v.at[0]],
                                  buf.at[pl.ds(0, CW)], sem).wait()
            return 0

        lax.fori_loop(0, nch, drain, 0)
        pltpu.sync_copy(buf, out.at[pl.ds(w * rows_per_tile, rows_per_tile)])

    return k


_sc_gather_edges = _make_sc_gather(NCH)
_sc_gather_pairs = _make_sc_gather(2 * B // NW // CW)


@functools.partial(
    pl.kernel,
    out_type=jax.ShapeDtypeStruct((TOT + NSLOP, D), jnp.float32),
    mesh=_MESH,
    compiler_params=pltpu.CompilerParams(use_tc_tiling_on_sc=False),
    scratch_types=[
        pltpu.VMEM_SHARED((SUMROWS, D), jnp.float32),
        pltpu.VMEM((SEG_CH, CW), jnp.int32),
        pltpu.VMEM((SEG_CH, CW), jnp.int32),
        pltpu.VMEM((RB + 1, CW, D), jnp.float32),
        pltpu.SemaphoreType.DMA((RB,)),
        pltpu.SemaphoreType.DMA((RB,)),
    ],
)
def _sc_segsum(msgs, zrows, tgt_all, pos_all, out, sums, idx_v, pos_v, ring,
               semL, semA):
    c = lax.axis_index("c")
    s = lax.axis_index("s")
    pltpu.sync_copy(tgt_all.at[c, s], idx_v)
    pltpu.sync_copy(pos_all.at[c, s], pos_v)
    base = s * SEG_ROWS

    pltpu.sync_copy(zrows, ring.at[RB])
    for b in range(RB):
        pltpu.async_copy(msgs.at[pl.ds(base + b * CW, CW)], ring.at[b],
                         semL.at[b])

    def zfire(kk, _):
        pltpu.async_copy(ring.at[RB], sums.at[idx_v.at[kk]], semA.at[0])
        return 0

    lax.fori_loop(0, SEG_CH, zfire, 0)

    def zdrain(kk, _):
        pltpu.make_async_copy(ring.at[RB], sums.at[idx_v.at[0]],
                              semA.at[0]).wait()
        return 0

    lax.fori_loop(0, SEG_CH, zdrain, 0)
    plsc.subcore_barrier()

    def agroup(g, _):
        for b in range(RB):
            k = g * RB + b
            pltpu.make_async_copy(msgs.at[pl.ds(base, CW)], ring.at[b],
                                  semL.at[b]).wait()
            pltpu.async_copy(ring.at[b], sums.at[idx_v.at[k]], semA.at[b],
                             add=True)
        for b in range(RB):
            k = g * RB + b

            @pl.when(g < NG - 1)
            def _prefire_next():
                pltpu.make_async_copy(ring.at[b], sums.at[idx_v.at[0]],
                                      semA.at[b]).wait()
                pltpu.async_copy(msgs.at[pl.ds(base + (k + RB) * CW, CW)],
                                 ring.at[b], semL.at[b])

        return 0

    lax.fori_loop(0, NG, agroup, 0)
    for b in range(RB):
        pltpu.make_async_copy(ring.at[b], sums.at[idx_v.at[0]],
                              semA.at[b]).wait()
    plsc.subcore_barrier()

    def bgroup(g, _):
        for b in range(RB):
            k = g * RB + b

            @pl.when(g > 0)
            def _wait_out():
                pltpu.make_async_copy(ring.at[b], out.at[pos_v.at[0]],
                                      semA.at[b]).wait()

            pltpu.async_copy(sums.at[idx_v.at[k]], ring.at[b], semL.at[b])
        for b in range(RB):
            k = g * RB + b
            pltpu.make_async_copy(sums.at[idx_v.at[0]], ring.at[b],
                                  semL.at[b]).wait()
            pltpu.async_copy(ring.at[b], out.at[pos_v.at[k]], semA.at[b])
        return 0

    lax.fori_loop(0, NG, bgroup, 0)
    for b in range(RB):
        pltpu.make_async_copy(ring.at[b], out.at[pos_v.at[0]],
                              semA.at[b]).wait()


CVW = 16


@functools.partial(
    pl.kernel,
    out_type=jax.ShapeDtypeStruct((TOT + NSLOP, CVW), jnp.float32),
    mesh=_MESH,
    compiler_params=pltpu.CompilerParams(use_tc_tiling_on_sc=False),
    scratch_types=[
        pltpu.VMEM_SHARED((SUMROWS, CVW), jnp.float32),
        pltpu.VMEM((SEG_CH, CW), jnp.int32),
        pltpu.VMEM((SEG_CH, CW), jnp.int32),
        pltpu.VMEM((RB, CW, CVW), jnp.float32),
        pltpu.SemaphoreType.DMA((RB,)),
        pltpu.SemaphoreType.DMA((RB,)),
    ],
)
def _sc_cnt(zrows, orows, tgt_all, pos_all, out, cnts, idx_v, pos_v, ring,
            semL, semA):
    c = lax.axis_index("c")
    s = lax.axis_index("s")
    pltpu.sync_copy(tgt_all.at[c, s], idx_v)
    pltpu.sync_copy(pos_all.at[c, s], pos_v)
    pltpu.sync_copy(zrows, ring.at[0])
    pltpu.sync_copy(orows, ring.at[1])

    def zfire(kk, _):
        pltpu.async_copy(ring.at[0], cnts.at[idx_v.at[kk]], semA.at[0])
        return 0

    lax.fori_loop(0, SEG_CH, zfire, 0)

    def zdrain(kk, _):
        pltpu.make_async_copy(ring.at[0], cnts.at[idx_v.at[0]],
                              semA.at[0]).wait()
        return 0

    lax.fori_loop(0, SEG_CH, zdrain, 0)
    plsc.subcore_barrier()

    def afire(kk, _):
        pltpu.async_copy(ring.at[1], cnts.at[idx_v.at[kk]], semA.at[1],
                         add=True)
        return 0

    lax.fori_loop(0, SEG_CH, afire, 0)

    def adrain(kk, _):
        pltpu.make_async_copy(ring.at[1], cnts.at[idx_v.at[0]],
                              semA.at[1]).wait()
        return 0

    lax.fori_loop(0, SEG_CH, adrain, 0)
    plsc.subcore_barrier()

    def bgroup(g, _):
        for b in range(RB):
            k = g * RB + b

            @pl.when(g > 0)
            def _wait_out():
                pltpu.make_async_copy(ring.at[b], out.at[pos_v.at[0]],
                                      semA.at[b]).wait()

            pltpu.async_copy(cnts.at[idx_v.at[k]], ring.at[b], semL.at[b])
        for b in range(RB):
            k = g * RB + b
            pltpu.make_async_copy(cnts.at[idx_v.at[0]], ring.at[b],
                                  semL.at[b]).wait()
            pltpu.async_copy(ring.at[b], out.at[pos_v.at[k]], semA.at[b])
        return 0

    lax.fori_loop(0, NG, bgroup, 0)
    for b in range(RB):
        pltpu.make_async_copy(ring.at[b], out.at[pos_v.at[0]],
                              semA.at[b]).wait()


@functools.partial(
    pl.kernel,
    out_type=jax.ShapeDtypeStruct((XROWS, D), jnp.float32),
    mesh=_MESH,
    compiler_params=pltpu.CompilerParams(use_tc_tiling_on_sc=False),
    scratch_types=[
        pltpu.VMEM((HALF // NSUB, D), jnp.float32),
        pltpu.VMEM((SEG_CH, CW), jnp.int32),
        pltpu.VMEM((RB, CW, D), jnp.float32),
        pltpu.SemaphoreType.DMA((RB,)),
        pltpu.SemaphoreType.DMA((RB,)),
    ],
)
def _sc_buildx(emb, upd, xtgt_all, x, cbuf, idx_v, ring, semL, semA):
    c = lax.axis_index("c")
    s = lax.axis_index("s")
    tbase = c * HALF + s * (HALF // NSUB)
    pltpu.sync_copy(emb.at[pl.ds(tbase, HALF // NSUB)], cbuf)
    pltpu.sync_copy(cbuf, x.at[pl.ds(tbase, HALF // NSUB)])
    plsc.subcore_barrier()

    pltpu.sync_copy(xtgt_all.at[c, s], idx_v)
    base = s * SEG_ROWS

    def group(g, _):
        for b in range(RB):
            k = g * RB + b

            @pl.when(g > 0)
            def _wait_scat():
                pltpu.make_async_copy(ring.at[b], x.at[idx_v.at[0]],
                                      semA.at[b]).wait()

            pltpu.async_copy(upd.at[pl.ds(base + k * CW, CW)], ring.at[b],
                             semL.at[b])
        for b in range(RB):
            k = g * RB + b
            pltpu.make_async_copy(upd.at[pl.ds(base, CW)], ring.at[b],
                                  semL.at[b]).wait()
            pltpu.async_copy(ring.at[b], x.at[idx_v.at[k]], semA.at[b])
        return 0

    lax.fori_loop(0, NG, group, 0)
    for b in range(RB):
        pltpu.make_async_copy(ring.at[b], x.at[idx_v.at[0]], semA.at[b]).wait()


PG = 4
PR = TOT // PG
PBLK = BLK // PG
NPB = PR // PBLK
HPB = (EP // PG) // PBLK


def _alt_pair(W, n_out):
    a = W.shape[0] // 2
    top, bot = W[:a], W[a:]
    Z = jnp.zeros((PG * 2 * a, PG * n_out), jnp.float32)
    for g in range(PG):
        Z = Z.at[g * a:(g + 1) * a, g * n_out:(g + 1) * n_out].set(top)
        Z = Z.at[PG * a + g * a:PG * a + (g + 1) * a,
                 g * n_out:(g + 1) * n_out].set(bot)
    return Z


def _bdiag(W):
    k, n = W.shape
    Z = jnp.zeros((PG * k, PG * n), jnp.float32)
    for g in range(PG):
        Z = Z.at[g * k:(g + 1) * k, g * n:(g + 1) * n].set(W)
    return Z


def _spec_ap():
    return pl.BlockSpec((PBLK, PG * D), lambda i: (i, 0))


def _spec_bp():
    return pl.BlockSpec((PBLK, PG * D), lambda i: ((i + HPB) % NPB, 0))


def _wspec(shape):
    nd = len(shape)
    return pl.BlockSpec(shape, lambda i: (0,) * nd)


def _tc_msgs_body(xa, xb, w1, b1, w2, b2, o):
    a = jnp.concatenate([xa[...], xb[...]], axis=1)
    h = jnp.maximum(jnp.dot(a, w1[...], preferred_element_type=jnp.float32)
                    + b1[...], 0.0)
    o[...] = jnp.dot(h, w2[...], preferred_element_type=jnp.float32) + b2[...]


def _tc_msgs(rows_p, W1a, b1t, W2d, b2t, swapped):
    first, second = ((_spec_bp(), _spec_ap()) if swapped
                     else (_spec_ap(), _spec_bp()))
    return pl.pallas_call(
        _tc_msgs_body,
        grid=(NPB,),
        in_specs=[first, second, _wspec((PG * 2 * D, PG * H)),
                  _wspec((1, PG * H)), _wspec((PG * H, PG * D)),
                  _wspec((1, PG * D))],
        out_specs=pl.BlockSpec((PBLK, PG * D), lambda i: (i, 0)),
        out_shape=jax.ShapeDtypeStruct((PR, PG * D), jnp.float32),
    )(rows_p, rows_p, W1a, b1t, W2d, b2t)


def _tc_upd_body(xt, sv, r, wu, bu, o):
    a = jnp.concatenate([xt[...], sv[...] * r[...]], axis=1)
    o[...] = jnp.maximum(
        jnp.dot(a, wu[...], preferred_element_type=jnp.float32) + bu[...], 0.0)


def _tc_upd(rows_p, sv_p, r_p, Wua, but, swapped):
    xt_spec = _spec_ap() if swapped else _spec_bp()
    return pl.pallas_call(
        _tc_upd_body,
        grid=(NPB,),
        in_specs=[xt_spec, pl.BlockSpec((PBLK, PG * D), lambda i: (i, 0)),
                  _spec_ap(), _wspec((PG * 2 * D, PG * D)),
                  _wspec((1, PG * D))],
        out_specs=pl.BlockSpec((PBLK, PG * D), lambda i: (i, 0)),
        out_shape=jax.ShapeDtypeStruct((PR, PG * D), jnp.float32),
    )(rows_p, sv_p, r_p, Wua, but)


def _tc_pred_body(he, ae, w1, b1, w2, b2, w3, b3, o):
    a = jnp.concatenate([he[...], ae[...]], axis=1)
    p = jnp.maximum(jnp.dot(a, w1[...], preferred_element_type=jnp.float32)
                    + b1[...], 0.0)
    p = jnp.maximum(jnp.dot(p, w2[...], preferred_element_type=jnp.float32)
                    + b2[...], 0.0)
    z = jnp.dot(p, w3[...], preferred_element_type=jnp.float32) + b3[...]
    o[...] = 1.0 / (1.0 + jnp.exp(-z))


def _tc_pred(ha_p, Wp1a, bp1t, Wp2d, bp2t, Wp3d, bp3t):
    npb = (B // PG) // PBLK
    he_spec = pl.BlockSpec((PBLK, PG * D), lambda i: (i, 0))
    ae_spec = pl.BlockSpec((PBLK, PG * D), lambda i: (i + npb, 0))
    return pl.pallas_call(
        _tc_pred_body,
        grid=(npb,),
        in_specs=[he_spec, ae_spec, _wspec((PG * 2 * D, PG * H)),
                  _wspec((1, PG * H)), _wspec((PG * H, PG * D)),
                  _wspec((1, PG * D)), _wspec((PG * D, PG)),
                  _wspec((1, PG))],
        out_specs=pl.BlockSpec((PBLK, PG), lambda i: (i, 0)),
        out_shape=jax.ShapeDtypeStruct((B // PG, PG), jnp.float32),
    )(ha_p, ha_p, Wp1a, bp1t, Wp2d, bp2t, Wp3d, bp3t)


def kernel(emb, W1, b1, W2, b2, Wu, bu, Wp1, bp1, Wp2, bp2, Wp3, bp3,
           home_ids, away_ids, adjacency):
    adjacency = adjacency.astype(jnp.int32)
    home_ids = home_ids.astype(jnp.int32)
    away_ids = away_ids.astype(jnp.int32)
    src = adjacency[:, 0]
    dst = adjacency[:, 1]

    pad0 = (jnp.arange(EP - E, dtype=jnp.int32) * 523) % N_TEAMS
    padm = jnp.full((EP - E,), -1, jnp.int32)
    idxcat = jnp.concatenate([src, pad0, dst, pad0])
    tgt = jnp.concatenate([dst, padm, src, padm])

    in0 = (tgt >= 0) & (tgt < HALF)
    in1 = tgt >= HALF
    pos = jnp.arange(TOT, dtype=jnp.int32)
    slop = pos & (NSLOP - 1)
    tgt_all = jnp.stack([jnp.where(in0, tgt, DUMP + slop),
                         jnp.where(in1, tgt - HALF, DUMP + slop)])
    tgt_all = tgt_all.reshape(2, NSUB, SEG_CH, CW)
    pos_all = jnp.stack([jnp.where(in0, pos, OUT_DUMP + slop),
                         jnp.where(in1, pos, OUT_DUMP + slop)])
    pos_all = pos_all.reshape(2, NSUB, SEG_CH, CW)
    xtgt_all = jnp.stack([jnp.where(in0, tgt, XDUMP + slop),
                          jnp.where(in1, tgt, XDUMP + slop)])
    xtgt_all = xtgt_all.reshape(2, NSUB, SEG_CH, CW)
    idx3 = idxcat.reshape(NW, NCH, CW)
    ha3 = jnp.concatenate([home_ids, away_ids]).reshape(NW, 2 * B // NW // CW, CW)

    zrows = jnp.zeros((CW, D), jnp.float32)
    zrows16 = jnp.zeros((CW, CVW), jnp.float32)
    orows16 = jnp.ones((CW, CVW), jnp.float32)
    W1a = _alt_pair(W1, H)
    b1t = jnp.tile(b1, PG).reshape(1, PG * H)
    W2d = _bdiag(W2)
    b2t = jnp.tile(b2, PG).reshape(1, PG * D)
    Wua = _alt_pair(Wu, D)
    but = jnp.tile(bu, PG).reshape(1, PG * D)
    Wp1a = _alt_pair(Wp1, H)
    bp1t = jnp.tile(bp1, PG).reshape(1, PG * H)
    Wp2d = _bdiag(Wp2)
    bp2t = jnp.tile(bp2, PG).reshape(1, PG * D)
    Wp3d = _bdiag(Wp3)
    bp3t = jnp.tile(bp3, PG).reshape(1, PG)

    rows = _sc_gather_edges(emb, idx3)
    cv = _sc_cnt(zrows16, orows16, tgt_all, pos_all)
    r_p = jnp.broadcast_to(1.0 / jnp.maximum(cv[:TOT, 0:1], 1.0),
                           (TOT, D)).reshape(PR, PG * D)

    rows_p = rows.reshape(PR, PG * D)
    swapped = False
    for _ in range(PASSES):
        msgs_p = _tc_msgs(rows_p, W1a, b1t, W2d, b2t, swapped)
        sv = _sc_segsum(msgs_p.reshape(TOT, D), zrows, tgt_all, pos_all)
        sv_p = sv.reshape((TOT + NSLOP) // PG, PG * D)
        rows_p = _tc_upd(rows_p, sv_p, r_p, Wua, but, swapped)
        swapped = True

    x = _sc_buildx(emb, rows_p.reshape(TOT, D), xtgt_all)
    rows_ha = _sc_gather_pairs(x, ha3)
    ha_p = rows_ha.reshape(2 * B // PG, PG * D)
    return _tc_pred(ha_p, Wp1a, bp1t, Wp2d, bp2t, Wp3d, bp3t).reshape(B, 1)


# speedup vs baseline: 7.6186x; 1.0742x over previous
import functools

import jax
import jax.numpy as jnp
from jax import lax
from jax.experimental import pallas as pl
from jax.experimental.pallas import tpu as pltpu
from jax.experimental.pallas import tpu_sc as plsc

N_TEAMS = 100000
D = 32
H = 64
E = 20000
B = 16384
PASSES = 3

EP = 20480
TOT = 2 * EP
NW = 32
NSUB = 16
HALF = 50000
NSLOP = 256
DUMP = HALF
SUMROWS = HALF + NSLOP
OUT_DUMP = TOT
XDUMP = N_TEAMS
XROWS = N_TEAMS + NSLOP
CW = 128
NCH = TOT // NW // CW
CHUNK = NCH * CW
SEG_CH = TOT // NSUB // CW
SEG_ROWS = SEG_CH * CW
RB = 5
NG = SEG_CH // RB

_MESH = plsc.VectorSubcoreMesh(core_axis_name="c", subcore_axis_name="s")

BLK = 2048
NB = TOT // BLK
HB = EP // BLK


def _make_sc_gather(nch):
    rows_per_tile = nch * CW

    @functools.partial(
        pl.kernel,
        out_type=jax.ShapeDtypeStruct((NW * rows_per_tile, D), jnp.float32),
        mesh=_MESH,
        compiler_params=pltpu.CompilerParams(use_tc_tiling_on_sc=False),
        scratch_types=[
            pltpu.VMEM((nch, CW), jnp.int32),
            pltpu.VMEM((rows_per_tile, D), jnp.float32),
            pltpu.SemaphoreType.DMA,
        ],
    )
    def k(table, idx3, out, idx_v, buf, sem):
        c = lax.axis_index("c")
        s = lax.axis_index("s")
        w = c * NSUB + s
        pltpu.sync_copy(idx3.at[w], idx_v)

        def fire(kk, _):
            pltpu.async_copy(table.at[idx_v.at[kk]],
                             buf.at[pl.ds(kk * CW, CW)], sem)
            return 0

        lax.fori_loop(0, nch, fire, 0)

        def drain(kk, _):
            pltpu.make_async_copy(table.at[idx_v.at[0]],
                                  buf.at[pl.ds(0, CW)], sem).wait()
            return 0

        lax.fori_loop(0, nch, drain, 0)
        pltpu.sync_copy(buf, out.at[pl.ds(w * rows_per_tile, rows_per_tile)])

    return k


_sc_gather_edges = _make_sc_gather(NCH)
_sc_gather_pairs = _make_sc_gather(2 * B // NW // CW)


@functools.partial(
    pl.kernel,
    out_type=jax.ShapeDtypeStruct((TOT + NSLOP, D), jnp.float32),
    mesh=_MESH,
    compiler_params=pltpu.CompilerParams(use_tc_tiling_on_sc=False),
    scratch_types=[
        pltpu.VMEM_SHARED((SUMROWS, D), jnp.float32),
        pltpu.VMEM((SEG_CH, CW), jnp.int32),
        pltpu.VMEM((SEG_CH, CW), jnp.int32),
        pltpu.VMEM((RB + 1, CW, D), jnp.float32),
        pltpu.SemaphoreType.DMA((RB,)),
        pltpu.SemaphoreType.DMA((RB,)),
    ],
)
def _sc_segsum(msgs, zrows, tgt_all, pos_all, out, sums, idx_v, pos_v, ring,
               semL, semA):
    c = lax.axis_index("c")
    s = lax.axis_index("s")
    pltpu.sync_copy(tgt_all.at[c, s], idx_v)
    pltpu.sync_copy(pos_all.at[c, s], pos_v)
    base = s * SEG_ROWS

    pltpu.sync_copy(zrows, ring.at[RB])
    for b in range(RB):
        pltpu.async_copy(msgs.at[pl.ds(base + b * CW, CW)], ring.at[b],
                         semL.at[b])

    def zfire(kk, _):
        pltpu.async_copy(ring.at[RB], sums.at[idx_v.at[kk]], semA.at[0])
        return 0

    lax.fori_loop(0, SEG_CH, zfire, 0)

    def zdrain(kk, _):
        pltpu.make_async_copy(ring.at[RB], sums.at[idx_v.at[0]],
                              semA.at[0]).wait()
        return 0

    lax.fori_loop(0, SEG_CH, zdrain, 0)
    plsc.subcore_barrier()

    def agroup(g, _):
        for b in range(RB):
            k = g * RB + b
            pltpu.make_async_copy(msgs.at[pl.ds(base, CW)], ring.at[b],
                                  semL.at[b]).wait()
            pltpu.async_copy(ring.at[b], sums.at[idx_v.at[k]], semA.at[b],
                             add=True)
        for b in range(RB):
            k = g * RB + b

            @pl.when(g < NG - 1)
            def _prefire_next():
                pltpu.make_async_copy(ring.at[b], sums.at[idx_v.at[0]],
                                      semA.at[b]).wait()
                pltpu.async_copy(msgs.at[pl.ds(base + (k + RB) * CW, CW)],
                                 ring.at[b], semL.at[b])

        return 0

    lax.fori_loop(0, NG, agroup, 0)
    for b in range(RB):
        pltpu.make_async_copy(ring.at[b], sums.at[idx_v.at[0]],
                              semA.at[b]).wait()
    plsc.subcore_barrier()

    def bgroup(g, _):
        for b in range(RB):
            k = g * RB + b

            @pl.when(g > 0)
            def _wait_out():
                pltpu.make_async_copy(ring.at[b], out.at[pos_v.at[0]],
                                      semA.at[b]).wait()

            pltpu.async_copy(sums.at[idx_v.at[k]], ring.at[b], semL.at[b])
        for b in range(RB):
            k = g * RB + b
            pltpu.make_async_copy(sums.at[idx_v.at[0]], ring.at[b],
                                  semL.at[b]).wait()
            pltpu.async_copy(ring.at[b], out.at[pos_v.at[k]], semA.at[b])
        return 0

    lax.fori_loop(0, NG, bgroup, 0)
    for b in range(RB):
        pltpu.make_async_copy(ring.at[b], out.at[pos_v.at[0]],
                              semA.at[b]).wait()


CVW = 16


@functools.partial(
    pl.kernel,
    out_type=jax.ShapeDtypeStruct((TOT + NSLOP, CVW), jnp.float32),
    mesh=_MESH,
    compiler_params=pltpu.CompilerParams(use_tc_tiling_on_sc=False),
    scratch_types=[
        pltpu.VMEM_SHARED((SUMROWS, CVW), jnp.float32),
        pltpu.VMEM((SEG_CH, CW), jnp.int32),
        pltpu.VMEM((SEG_CH, CW), jnp.int32),
        pltpu.VMEM((RB, CW, CVW), jnp.float32),
        pltpu.SemaphoreType.DMA((RB,)),
        pltpu.SemaphoreType.DMA((RB,)),
    ],
)
def _sc_cnt(zrows, orows, tgt_all, pos_all, out, cnts, idx_v, pos_v, ring,
            semL, semA):
    c = lax.axis_index("c")
    s = lax.axis_index("s")
    pltpu.sync_copy(tgt_all.at[c, s], idx_v)
    pltpu.sync_copy(pos_all.at[c, s], pos_v)
    pltpu.sync_copy(zrows, ring.at[0])
    pltpu.sync_copy(orows, ring.at[1])

    def zfire(kk, _):
        pltpu.async_copy(ring.at[0], cnts.at[idx_v.at[kk]], semA.at[0])
        return 0

    lax.fori_loop(0, SEG_CH, zfire, 0)

    def zdrain(kk, _):
        pltpu.make_async_copy(ring.at[0], cnts.at[idx_v.at[0]],
                              semA.at[0]).wait()
        return 0

    lax.fori_loop(0, SEG_CH, zdrain, 0)
    plsc.subcore_barrier()

    def afire(kk, _):
        pltpu.async_copy(ring.at[1], cnts.at[idx_v.at[kk]], semA.at[1],
                         add=True)
        return 0

    lax.fori_loop(0, SEG_CH, afire, 0)

    def adrain(kk, _):
        pltpu.make_async_copy(ring.at[1], cnts.at[idx_v.at[0]],
                              semA.at[1]).wait()
        return 0

    lax.fori_loop(0, SEG_CH, adrain, 0)
    plsc.subcore_barrier()

    def bgroup(g, _):
        for b in range(RB):
            k = g * RB + b

            @pl.when(g > 0)
            def _wait_out():
                pltpu.make_async_copy(ring.at[b], out.at[pos_v.at[0]],
                                      semA.at[b]).wait()

            pltpu.async_copy(cnts.at[idx_v.at[k]], ring.at[b], semL.at[b])
        for b in range(RB):
            k = g * RB + b
            pltpu.make_async_copy(cnts.at[idx_v.at[0]], ring.at[b],
                                  semL.at[b]).wait()
            pltpu.async_copy(ring.at[b], out.at[pos_v.at[k]], semA.at[b])
        return 0

    lax.fori_loop(0, NG, bgroup, 0)
    for b in range(RB):
        pltpu.make_async_copy(ring.at[b], out.at[pos_v.at[0]],
                              semA.at[b]).wait()


@functools.partial(
    pl.kernel,
    out_type=jax.ShapeDtypeStruct((XROWS, D), jnp.float32),
    mesh=_MESH,
    compiler_params=pltpu.CompilerParams(use_tc_tiling_on_sc=False),
    scratch_types=[
        pltpu.VMEM((HALF // NSUB, D), jnp.float32),
        pltpu.VMEM((SEG_CH, CW), jnp.int32),
        pltpu.VMEM((RB, CW, D), jnp.float32),
        pltpu.SemaphoreType.DMA((RB,)),
        pltpu.SemaphoreType.DMA((RB,)),
    ],
)
def _sc_buildx(emb, upd, xtgt_all, x, cbuf, idx_v, ring, semL, semA):
    c = lax.axis_index("c")
    s = lax.axis_index("s")
    tbase = c * HALF + s * (HALF // NSUB)
    pltpu.sync_copy(emb.at[pl.ds(tbase, HALF // NSUB)], cbuf)
    pltpu.sync_copy(cbuf, x.at[pl.ds(tbase, HALF // NSUB)])
    plsc.subcore_barrier()

    pltpu.sync_copy(xtgt_all.at[c, s], idx_v)
    base = s * SEG_ROWS

    def group(g, _):
        for b in range(RB):
            k = g * RB + b

            @pl.when(g > 0)
            def _wait_scat():
                pltpu.make_async_copy(ring.at[b], x.at[idx_v.at[0]],
                                      semA.at[b]).wait()

            pltpu.async_copy(upd.at[pl.ds(base + k * CW, CW)], ring.at[b],
                             semL.at[b])
        for b in range(RB):
            k = g * RB + b
            pltpu.make_async_copy(upd.at[pl.ds(base, CW)], ring.at[b],
                                  semL.at[b]).wait()
            pltpu.async_copy(ring.at[b], x.at[idx_v.at[k]], semA.at[b])
        return 0

    lax.fori_loop(0, NG, group, 0)
    for b in range(RB):
        pltpu.make_async_copy(ring.at[b], x.at[idx_v.at[0]], semA.at[b]).wait()


PG = 4
PR = TOT // PG
PBLK = BLK // PG
NPB = PR // PBLK
HPB = (EP // PG) // PBLK


def _alt_pair(W, n_out):
    a = W.shape[0] // 2
    top, bot = W[:a], W[a:]
    Z = jnp.zeros((PG * 2 * a, PG * n_out), jnp.float32)
    for g in range(PG):
        Z = Z.at[g * a:(g + 1) * a, g * n_out:(g + 1) * n_out].set(top)
        Z = Z.at[PG * a + g * a:PG * a + (g + 1) * a,
                 g * n_out:(g + 1) * n_out].set(bot)
    return Z


def _bdiag(W):
    k, n = W.shape
    Z = jnp.zeros((PG * k, PG * n), jnp.float32)
    for g in range(PG):
        Z = Z.at[g * k:(g + 1) * k, g * n:(g + 1) * n].set(W)
    return Z


def _spec_ap():
    return pl.BlockSpec((PBLK, PG * D), lambda i: (i, 0))


def _spec_bp():
    return pl.BlockSpec((PBLK, PG * D), lambda i: ((i + HPB) % NPB, 0))


def _wspec(shape):
    nd = len(shape)
    return pl.BlockSpec(shape, lambda i: (0,) * nd)


def _tc_msgs_body(xa, xb, w1, b1, w2, b2, o):
    a = jnp.concatenate([xa[...], xb[...]], axis=1)
    h = jnp.maximum(jnp.dot(a, w1[...], preferred_element_type=jnp.float32)
                    + b1[...], 0.0)
    o[...] = jnp.dot(h, w2[...], preferred_element_type=jnp.float32) + b2[...]


def _tc_msgs(rows_p, W1a, b1t, W2d, b2t, swapped):
    first, second = ((_spec_bp(), _spec_ap()) if swapped
                     else (_spec_ap(), _spec_bp()))
    return pl.pallas_call(
        _tc_msgs_body,
        grid=(NPB,),
        in_specs=[first, second, _wspec((PG * 2 * D, PG * H)),
                  _wspec((1, PG * H)), _wspec((PG * H, PG * D)),
                  _wspec((1, PG * D))],
        out_specs=pl.BlockSpec((PBLK, PG * D), lambda i: (i, 0)),
        out_shape=jax.ShapeDtypeStruct((PR, PG * D), jnp.float32),
    )(rows_p, rows_p, W1a, b1t, W2d, b2t)


def _tc_upd_body(xt, sv, r, wu, bu, o):
    a = jnp.concatenate([xt[...], sv[...] * r[...]], axis=1)
    o[...] = jnp.maximum(
        jnp.dot(a, wu[...], preferred_element_type=jnp.float32) + bu[...], 0.0)


def _tc_upd(rows_p, sv_p, r_p, Wua, but, swapped):
    xt_spec = _spec_ap() if swapped else _spec_bp()
    return pl.pallas_call(
        _tc_upd_body,
        grid=(NPB,),
        in_specs=[xt_spec, pl.BlockSpec((PBLK, PG * D), lambda i: (i, 0)),
                  _spec_ap(), _wspec((PG * 2 * D, PG * D)),
                  _wspec((1, PG * D))],
        out_specs=pl.BlockSpec((PBLK, PG * D), lambda i: (i, 0)),
        out_shape=jax.ShapeDtypeStruct((PR, PG * D), jnp.float32),
    )(rows_p, sv_p, r_p, Wua, but)


def _tc_pred_body(he, ae, w1, b1, w2, b2, w3, b3, o):
    a = jnp.concatenate([he[...], ae[...]], axis=1)
    p = jnp.maximum(jnp.dot(a, w1[...], preferred_element_type=jnp.float32)
                    + b1[...], 0.0)
    p = jnp.maximum(jnp.dot(p, w2[...], preferred_element_type=jnp.float32)
                    + b2[...], 0.0)
    z = jnp.dot(p, w3[...], preferred_element_type=jnp.float32) + b3[...]
    o[...] = 1.0 / (1.0 + jnp.exp(-z))


def _tc_pred(ha_p, Wp1a, bp1t, Wp2d, bp2t, Wp3d, bp3t):
    npb = (B // PG) // PBLK
    he_spec = pl.BlockSpec((PBLK, PG * D), lambda i: (i, 0))
    ae_spec = pl.BlockSpec((PBLK, PG * D), lambda i: (i + npb, 0))
    return pl.pallas_call(
        _tc_pred_body,
        grid=(npb,),
        in_specs=[he_spec, ae_spec, _wspec((PG * 2 * D, PG * H)),
                  _wspec((1, PG * H)), _wspec((PG * H, PG * D)),
                  _wspec((1, PG * D)), _wspec((PG * D, PG)),
                  _wspec((1, PG))],
        out_specs=pl.BlockSpec((PBLK, PG), lambda i: (i, 0)),
        out_shape=jax.ShapeDtypeStruct((B // PG, PG), jnp.float32),
    )(ha_p, ha_p, Wp1a, bp1t, Wp2d, bp2t, Wp3d, bp3t)


def kernel(emb, W1, b1, W2, b2, Wu, bu, Wp1, bp1, Wp2, bp2, Wp3, bp3,
           home_ids, away_ids, adjacency):
    adjacency = adjacency.astype(jnp.int32)
    home_ids = home_ids.astype(jnp.int32)
    away_ids = away_ids.astype(jnp.int32)
    src = adjacency[:, 0]
    dst = adjacency[:, 1]

    pad0 = (jnp.arange(EP - E, dtype=jnp.int32) * 523) % N_TEAMS
    padm = jnp.full((EP - E,), -1, jnp.int32)
    idxcat = jnp.concatenate([src, pad0, dst, pad0])
    tgt = jnp.concatenate([dst, padm, src, padm])

    in0 = (tgt >= 0) & (tgt < HALF)
    in1 = tgt >= HALF
    pos = jnp.arange(TOT, dtype=jnp.int32)
    slop = pos & (NSLOP - 1)
    tgt_all = jnp.stack([jnp.where(in0, tgt, DUMP + slop),
                         jnp.where(in1, tgt - HALF, DUMP + slop)])
    tgt_all = tgt_all.reshape(2, NSUB, SEG_CH, CW)
    pos_all = jnp.stack([jnp.where(in0, pos, OUT_DUMP + slop),
                         jnp.where(in1, pos, OUT_DUMP + slop)])
    pos_all = pos_all.reshape(2, NSUB, SEG_CH, CW)
    xtgt_all = jnp.stack([jnp.where(in0, tgt, XDUMP + slop),
                          jnp.where(in1, tgt, XDUMP + slop)])
    xtgt_all = xtgt_all.reshape(2, NSUB, SEG_CH, CW)
    idx3 = idxcat.reshape(NW, NCH, CW)
    ha3 = jnp.concatenate([home_ids, away_ids]).reshape(NW, 2 * B // NW // CW, CW)

    zrows = jnp.zeros((CW, D), jnp.float32)
    zrows16 = jnp.zeros((CW, CVW), jnp.float32)
    orows16 = jnp.ones((CW, CVW), jnp.float32)
    W1a = _alt_pair(W1, H)
    b1t = jnp.tile(b1, PG).reshape(1, PG * H)
    W2d = _bdiag(W2)
    b2t = jnp.tile(b2, PG).reshape(1, PG * D)
    Wua = _alt_pair(Wu, D)
    but = jnp.tile(bu, PG).reshape(1, PG * D)
    Wp1a = _alt_pair(Wp1, H)
    bp1t = jnp.tile(bp1, PG).reshape(1, PG * H)
    Wp2d = _bdiag(Wp2)
    bp2t = jnp.tile(bp2, PG).reshape(1, PG * D)
    Wp3d = _bdiag(Wp3)
    bp3t = jnp.tile(bp3, PG).reshape(1, PG)

    rows = _sc_gather_edges(emb, idx3)
    cv = _sc_cnt(zrows16, orows16, tgt_all, pos_all)
    r_p = jnp.broadcast_to(1.0 / jnp.maximum(cv[:TOT, 0:1], 1.0),
                           (TOT, D)).reshape(PR, PG * D)

    rows_p = rows.reshape(PR, PG * D)
    swapped = False
    for _ in range(PASSES):
        msgs_p = _tc_msgs(rows_p, W1a, b1t, W2d, b2t, swapped)
        sv = _sc_segsum(msgs_p.reshape(TOT, D), zrows, tgt_all, pos_all)
        sv_p = sv.reshape((TOT + NSLOP) // PG, PG * D)
        rows_p = _tc_upd(rows_p, sv_p, r_p, Wua, but, swapped)
        swapped = True

    x = _sc_buildx(emb, rows_p.reshape(TOT, D), xtgt_all)
    rows_ha = _sc_gather_pairs(x, ha3)
    ha_p = rows_ha.reshape(2 * B // PG, PG * D)
    return _tc_pred(ha_p, Wp1a, bp1t, Wp2d, bp2t, Wp3d, bp3t).reshape(B, 1)


# speedup vs baseline: 7.6213x; 1.0003x over previous
import functools

import jax
import jax.numpy as jnp
from jax import lax
from jax.experimental import pallas as pl
from jax.experimental.pallas import tpu as pltpu
from jax.experimental.pallas import tpu_sc as plsc

N_TEAMS = 100000
D = 32
H = 64
E = 20000
B = 16384
PASSES = 3

EP = 20480
TOT = 2 * EP
NW = 32
NSUB = 16
HALF = 50000
NSLOP = 512
DUMP = HALF
SUMROWS = HALF + NSLOP
OUT_DUMP = TOT
XDUMP = N_TEAMS
XROWS = N_TEAMS + NSLOP
CW = 128
NCH = TOT // NW // CW
CHUNK = NCH * CW
SEG_CH = TOT // NSUB // CW
SEG_ROWS = SEG_CH * CW
RB = 5
NG = SEG_CH // RB

_MESH = plsc.VectorSubcoreMesh(core_axis_name="c", subcore_axis_name="s")

BLK = 2048
NB = TOT // BLK
HB = EP // BLK


def _make_sc_gather(nch):
    rows_per_tile = nch * CW

    @functools.partial(
        pl.kernel,
        out_type=jax.ShapeDtypeStruct((NW * rows_per_tile, D), jnp.float32),
        mesh=_MESH,
        compiler_params=pltpu.CompilerParams(use_tc_tiling_on_sc=False),
        scratch_types=[
            pltpu.VMEM((nch, CW), jnp.int32),
            pltpu.VMEM((rows_per_tile, D), jnp.float32),
            pltpu.SemaphoreType.DMA,
        ],
    )
    def k(table, idx3, out, idx_v, buf, sem):
        c = lax.axis_index("c")
        s = lax.axis_index("s")
        w = c * NSUB + s
        pltpu.sync_copy(idx3.at[w], idx_v)

        def fire(kk, _):
            pltpu.async_copy(table.at[idx_v.at[kk]],
                             buf.at[pl.ds(kk * CW, CW)], sem)
            return 0

        lax.fori_loop(0, nch, fire, 0)

        def drain(kk, _):
            pltpu.make_async_copy(table.at[idx_v.at[0]],
                                  buf.at[pl.ds(0, CW)], sem).wait()
            return 0

        lax.fori_loop(0, nch, drain, 0)
        pltpu.sync_copy(buf, out.at[pl.ds(w * rows_per_tile, rows_per_tile)])

    return k


_sc_gather_edges = _make_sc_gather(NCH)
_sc_gather_pairs = _make_sc_gather(2 * B // NW // CW)


@functools.partial(
    pl.kernel,
    out_type=jax.ShapeDtypeStruct((TOT + NSLOP, D), jnp.float32),
    mesh=_MESH,
    compiler_params=pltpu.CompilerParams(use_tc_tiling_on_sc=False),
    scratch_types=[
        pltpu.VMEM_SHARED((SUMROWS, D), jnp.float32),
        pltpu.VMEM((SEG_CH, CW), jnp.int32),
        pltpu.VMEM((SEG_CH, CW), jnp.int32),
        pltpu.VMEM((RB + 1, CW, D), jnp.float32),
        pltpu.SemaphoreType.DMA((RB,)),
        pltpu.SemaphoreType.DMA((RB,)),
    ],
)
def _sc_segsum(msgs, zrows, tgt_all, pos_all, out, sums, idx_v, pos_v, ring,
               semL, semA):
    c = lax.axis_index("c")
    s = lax.axis_index("s")
    pltpu.sync_copy(tgt_all.at[c, s], idx_v)
    pltpu.sync_copy(pos_all.at[c, s], pos_v)
    base = s * SEG_ROWS

    pltpu.sync_copy(zrows, ring.at[RB])
    for b in range(RB):
        pltpu.async_copy(msgs.at[pl.ds(base + b * CW, CW)], ring.at[b],
                         semL.at[b])

    def zfire(kk, _):
        pltpu.async_copy(ring.at[RB], sums.at[idx_v.at[kk]], semA.at[0])
        return 0

    lax.fori_loop(0, SEG_CH, zfire, 0)

    def zdrain(kk, _):
        pltpu.make_async_copy(ring.at[RB], sums.at[idx_v.at[0]],
                              semA.at[0]).wait()
        return 0

    lax.fori_loop(0, SEG_CH, zdrain, 0)
    plsc.subcore_barrier()

    def agroup(g, _):
        for b in range(RB):
            k = g * RB + b
            pltpu.make_async_copy(msgs.at[pl.ds(base, CW)], ring.at[b],
                                  semL.at[b]).wait()
            pltpu.async_copy(ring.at[b], sums.at[idx_v.at[k]], semA.at[b],
                             add=True)
        for b in range(RB):
            k = g * RB + b

            @pl.when(g < NG - 1)
            def _prefire_next():
                pltpu.make_async_copy(ring.at[b], sums.at[idx_v.at[0]],
                                      semA.at[b]).wait()
                pltpu.async_copy(msgs.at[pl.ds(base + (k + RB) * CW, CW)],
                                 ring.at[b], semL.at[b])

        return 0

    lax.fori_loop(0, NG, agroup, 0)
    for b in range(RB):
        pltpu.make_async_copy(ring.at[b], sums.at[idx_v.at[0]],
                              semA.at[b]).wait()
    plsc.subcore_barrier()

    def bgroup(g, _):
        for b in range(RB):
            k = g * RB + b

            @pl.when(g > 0)
            def _wait_out():
                pltpu.make_async_copy(ring.at[b], out.at[pos_v.at[0]],
                                      semA.at[b]).wait()

            pltpu.async_copy(sums.at[idx_v.at[k]], ring.at[b], semL.at[b])
        for b in range(RB):
            k = g * RB + b
            pltpu.make_async_copy(sums.at[idx_v.at[0]], ring.at[b],
                                  semL.at[b]).wait()
            pltpu.async_copy(ring.at[b], out.at[pos_v.at[k]], semA.at[b])
        return 0

    lax.fori_loop(0, NG, bgroup, 0)
    for b in range(RB):
        pltpu.make_async_copy(ring.at[b], out.at[pos_v.at[0]],
                              semA.at[b]).wait()


CVW = 16


@functools.partial(
    pl.kernel,
    out_type=jax.ShapeDtypeStruct((TOT + NSLOP, CVW), jnp.float32),
    mesh=_MESH,
    compiler_params=pltpu.CompilerParams(use_tc_tiling_on_sc=False),
    scratch_types=[
        pltpu.VMEM_SHARED((SUMROWS, CVW), jnp.float32),
        pltpu.VMEM((SEG_CH, CW), jnp.int32),
        pltpu.VMEM((SEG_CH, CW), jnp.int32),
        pltpu.VMEM((RB, CW, CVW), jnp.float32),
        pltpu.SemaphoreType.DMA((RB,)),
        pltpu.SemaphoreType.DMA((RB,)),
    ],
)
def _sc_cnt(zrows, orows, tgt_all, pos_all, out, cnts, idx_v, pos_v, ring,
            semL, semA):
    c = lax.axis_index("c")
    s = lax.axis_index("s")
    pltpu.sync_copy(tgt_all.at[c, s], idx_v)
    pltpu.sync_copy(pos_all.at[c, s], pos_v)
    pltpu.sync_copy(zrows, ring.at[0])
    pltpu.sync_copy(orows, ring.at[1])

    def zfire(kk, _):
        pltpu.async_copy(ring.at[0], cnts.at[idx_v.at[kk]], semA.at[0])
        return 0

    lax.fori_loop(0, SEG_CH, zfire, 0)

    def zdrain(kk, _):
        pltpu.make_async_copy(ring.at[0], cnts.at[idx_v.at[0]],
                              semA.at[0]).wait()
        return 0

    lax.fori_loop(0, SEG_CH, zdrain, 0)
    plsc.subcore_barrier()

    def afire(kk, _):
        pltpu.async_copy(ring.at[1], cnts.at[idx_v.at[kk]], semA.at[1],
                         add=True)
        return 0

    lax.fori_loop(0, SEG_CH, afire, 0)

    def adrain(kk, _):
        pltpu.make_async_copy(ring.at[1], cnts.at[idx_v.at[0]],
                              semA.at[1]).wait()
        return 0

    lax.fori_loop(0, SEG_CH, adrain, 0)
    plsc.subcore_barrier()

    def bgroup(g, _):
        for b in range(RB):
            k = g * RB + b

            @pl.when(g > 0)
            def _wait_out():
                pltpu.make_async_copy(ring.at[b], out.at[pos_v.at[0]],
                                      semA.at[b]).wait()

            pltpu.async_copy(cnts.at[idx_v.at[k]], ring.at[b], semL.at[b])
        for b in range(RB):
            k = g * RB + b
            pltpu.make_async_copy(cnts.at[idx_v.at[0]], ring.at[b],
                                  semL.at[b]).wait()
            pltpu.async_copy(ring.at[b], out.at[pos_v.at[k]], semA.at[b])
        return 0

    lax.fori_loop(0, NG, bgroup, 0)
    for b in range(RB):
        pltpu.make_async_copy(ring.at[b], out.at[pos_v.at[0]],
                              semA.at[b]).wait()


@functools.partial(
    pl.kernel,
    out_type=jax.ShapeDtypeStruct((XROWS, D), jnp.float32),
    mesh=_MESH,
    compiler_params=pltpu.CompilerParams(use_tc_tiling_on_sc=False),
    scratch_types=[
        pltpu.VMEM((HALF // NSUB, D), jnp.float32),
        pltpu.VMEM((SEG_CH, CW), jnp.int32),
        pltpu.VMEM((RB, CW, D), jnp.float32),
        pltpu.SemaphoreType.DMA((RB,)),
        pltpu.SemaphoreType.DMA((RB,)),
    ],
)
def _sc_buildx(emb, upd, xtgt_all, x, cbuf, idx_v, ring, semL, semA):
    c = lax.axis_index("c")
    s = lax.axis_index("s")
    tbase = c * HALF + s * (HALF // NSUB)
    pltpu.sync_copy(emb.at[pl.ds(tbase, HALF // NSUB)], cbuf)
    pltpu.sync_copy(cbuf, x.at[pl.ds(tbase, HALF // NSUB)])
    plsc.subcore_barrier()

    pltpu.sync_copy(xtgt_all.at[c, s], idx_v)
    base = s * SEG_ROWS

    def group(g, _):
        for b in range(RB):
            k = g * RB + b

            @pl.when(g > 0)
            def _wait_scat():
                pltpu.make_async_copy(ring.at[b], x.at[idx_v.at[0]],
                                      semA.at[b]).wait()

            pltpu.async_copy(upd.at[pl.ds(base + k * CW, CW)], ring.at[b],
                             semL.at[b])
        for b in range(RB):
            k = g * RB + b
            pltpu.make_async_copy(upd.at[pl.ds(base, CW)], ring.at[b],
                                  semL.at[b]).wait()
            pltpu.async_copy(ring.at[b], x.at[idx_v.at[k]], semA.at[b])
        return 0

    lax.fori_loop(0, NG, group, 0)
    for b in range(RB):
        pltpu.make_async_copy(ring.at[b], x.at[idx_v.at[0]], semA.at[b]).wait()


PG = 4
PR = TOT // PG
PBLK = BLK // PG
NPB = PR // PBLK
HPB = (EP // PG) // PBLK


def _alt_pair(W, n_out):
    a = W.shape[0] // 2
    top, bot = W[:a], W[a:]
    Z = jnp.zeros((PG * 2 * a, PG * n_out), jnp.float32)
    for g in range(PG):
        Z = Z.at[g * a:(g + 1) * a, g * n_out:(g + 1) * n_out].set(top)
        Z = Z.at[PG * a + g * a:PG * a + (g + 1) * a,
                 g * n_out:(g + 1) * n_out].set(bot)
    return Z


def _bdiag(W):
    k, n = W.shape
    Z = jnp.zeros((PG * k, PG * n), jnp.float32)
    for g in range(PG):
        Z = Z.at[g * k:(g + 1) * k, g * n:(g + 1) * n].set(W)
    return Z


def _spec_ap():
    return pl.BlockSpec((PBLK, PG * D), lambda i: (i, 0))


def _spec_bp():
    return pl.BlockSpec((PBLK, PG * D), lambda i: ((i + HPB) % NPB, 0))


def _wspec(shape):
    nd = len(shape)
    return pl.BlockSpec(shape, lambda i: (0,) * nd)


def _tc_msgs_body(xa, xb, w1, b1, w2, b2, o):
    a = jnp.concatenate([xa[...], xb[...]], axis=1)
    h = jnp.maximum(jnp.dot(a, w1[...], preferred_element_type=jnp.float32)
                    + b1[...], 0.0)
    o[...] = jnp.dot(h, w2[...], preferred_element_type=jnp.float32) + b2[...]


def _tc_msgs(rows_p, W1a, b1t, W2d, b2t, swapped):
    first, second = ((_spec_bp(), _spec_ap()) if swapped
                     else (_spec_ap(), _spec_bp()))
    return pl.pallas_call(
        _tc_msgs_body,
        grid=(NPB,),
        in_specs=[first, second, _wspec((PG * 2 * D, PG * H)),
                  _wspec((1, PG * H)), _wspec((PG * H, PG * D)),
                  _wspec((1, PG * D))],
        out_specs=pl.BlockSpec((PBLK, PG * D), lambda i: (i, 0)),
        out_shape=jax.ShapeDtypeStruct((PR, PG * D), jnp.float32),
    )(rows_p, rows_p, W1a, b1t, W2d, b2t)


def _tc_upd_body(xt, sv, r, wu, bu, o):
    a = jnp.concatenate([xt[...], sv[...] * r[...]], axis=1)
    o[...] = jnp.maximum(
        jnp.dot(a, wu[...], preferred_element_type=jnp.float32) + bu[...], 0.0)


def _tc_upd(rows_p, sv_p, r_p, Wua, but, swapped):
    xt_spec = _spec_ap() if swapped else _spec_bp()
    return pl.pallas_call(
        _tc_upd_body,
        grid=(NPB,),
        in_specs=[xt_spec, pl.BlockSpec((PBLK, PG * D), lambda i: (i, 0)),
                  _spec_ap(), _wspec((PG * 2 * D, PG * D)),
                  _wspec((1, PG * D))],
        out_specs=pl.BlockSpec((PBLK, PG * D), lambda i: (i, 0)),
        out_shape=jax.ShapeDtypeStruct((PR, PG * D), jnp.float32),
    )(rows_p, sv_p, r_p, Wua, but)


def _tc_pred_body(he, ae, w1, b1, w2, b2, w3, b3, o):
    a = jnp.concatenate([he[...], ae[...]], axis=1)
    p = jnp.maximum(jnp.dot(a, w1[...], preferred_element_type=jnp.float32)
                    + b1[...], 0.0)
    p = jnp.maximum(jnp.dot(p, w2[...], preferred_element_type=jnp.float32)
                    + b2[...], 0.0)
    z = jnp.dot(p, w3[...], preferred_element_type=jnp.float32) + b3[...]
    o[...] = 1.0 / (1.0 + jnp.exp(-z))


def _tc_pred(ha_p, Wp1a, bp1t, Wp2d, bp2t, Wp3d, bp3t):
    npb = (B // PG) // PBLK
    he_spec = pl.BlockSpec((PBLK, PG * D), lambda i: (i, 0))
    ae_spec = pl.BlockSpec((PBLK, PG * D), lambda i: (i + npb, 0))
    return pl.pallas_call(
        _tc_pred_body,
        grid=(npb,),
        in_specs=[he_spec, ae_spec, _wspec((PG * 2 * D, PG * H)),
                  _wspec((1, PG * H)), _wspec((PG * H, PG * D)),
                  _wspec((1, PG * D)), _wspec((PG * D, PG)),
                  _wspec((1, PG))],
        out_specs=pl.BlockSpec((PBLK, PG), lambda i: (i, 0)),
        out_shape=jax.ShapeDtypeStruct((B // PG, PG), jnp.float32),
    )(ha_p, ha_p, Wp1a, bp1t, Wp2d, bp2t, Wp3d, bp3t)


def kernel(emb, W1, b1, W2, b2, Wu, bu, Wp1, bp1, Wp2, bp2, Wp3, bp3,
           home_ids, away_ids, adjacency):
    adjacency = adjacency.astype(jnp.int32)
    home_ids = home_ids.astype(jnp.int32)
    away_ids = away_ids.astype(jnp.int32)
    src = adjacency[:, 0]
    dst = adjacency[:, 1]

    pad0 = (jnp.arange(EP - E, dtype=jnp.int32) * 523) % N_TEAMS
    padm = jnp.full((EP - E,), -1, jnp.int32)
    idxcat = jnp.concatenate([src, pad0, dst, pad0])
    tgt = jnp.concatenate([dst, padm, src, padm])

    in0 = (tgt >= 0) & (tgt < HALF)
    in1 = tgt >= HALF
    pos = jnp.arange(TOT, dtype=jnp.int32)
    slop = pos & (NSLOP - 1)
    tgt_all = jnp.stack([jnp.where(in0, tgt, DUMP + slop),
                         jnp.where(in1, tgt - HALF, DUMP + slop)])
    tgt_all = tgt_all.reshape(2, NSUB, SEG_CH, CW)
    pos_all = jnp.stack([jnp.where(in0, pos, OUT_DUMP + slop),
                         jnp.where(in1, pos, OUT_DUMP + slop)])
    pos_all = pos_all.reshape(2, NSUB, SEG_CH, CW)
    xtgt_all = jnp.stack([jnp.where(in0, tgt, XDUMP + slop),
                          jnp.where(in1, tgt, XDUMP + slop)])
    xtgt_all = xtgt_all.reshape(2, NSUB, SEG_CH, CW)
    idx3 = idxcat.reshape(NW, NCH, CW)
    ha3 = jnp.concatenate([home_ids, away_ids]).reshape(NW, 2 * B // NW // CW, CW)

    zrows = jnp.zeros((CW, D), jnp.float32)
    zrows16 = jnp.zeros((CW, CVW), jnp.float32)
    orows16 = jnp.ones((CW, CVW), jnp.float32)
    W1a = _alt_pair(W1, H)
    b1t = jnp.tile(b1, PG).reshape(1, PG * H)
    W2d = _bdiag(W2)
    b2t = jnp.tile(b2, PG).reshape(1, PG * D)
    Wua = _alt_pair(Wu, D)
    but = jnp.tile(bu, PG).reshape(1, PG * D)
    Wp1a = _alt_pair(Wp1, H)
    bp1t = jnp.tile(bp1, PG).reshape(1, PG * H)
    Wp2d = _bdiag(Wp2)
    bp2t = jnp.tile(bp2, PG).reshape(1, PG * D)
    Wp3d = _bdiag(Wp3)
    bp3t = jnp.tile(bp3, PG).reshape(1, PG)

    rows = _sc_gather_edges(emb, idx3)
    cv = _sc_cnt(zrows16, orows16, tgt_all, pos_all)
    r_p = jnp.broadcast_to(1.0 / jnp.maximum(cv[:TOT, 0:1], 1.0),
                           (TOT, D)).reshape(PR, PG * D)

    rows_p = rows.reshape(PR, PG * D)
    swapped = False
    for _ in range(PASSES):
        msgs_p = _tc_msgs(rows_p, W1a, b1t, W2d, b2t, swapped)
        sv = _sc_segsum(msgs_p.reshape(TOT, D), zrows, tgt_all, pos_all)
        sv_p = sv.reshape((TOT + NSLOP) // PG, PG * D)
        rows_p = _tc_upd(rows_p, sv_p, r_p, Wua, but, swapped)
        swapped = True

    x = _sc_buildx(emb, rows_p.reshape(TOT, D), xtgt_all)
    rows_ha = _sc_gather_pairs(x, ha3)
    ha_p = rows_ha.reshape(2 * B // PG, PG * D)
    return _tc_pred(ha_p, Wp1a, bp1t, Wp2d, bp2t, Wp3d, bp3t).reshape(B, 1)
